# Initial kernel scaffold; baseline (speedup 1.0000x reference)
import jax
import jax.numpy as jnp
from jax.experimental import pallas as pl


def kernel(x, edge_index1, kidx1, parent1, edge_index2, kidx2, parent2, src_u1, dst_u1, kidx_u1, src_u2, dst_u2, kidx_u2, W1, W2, Wt1, Wt2):
    raise NotImplementedError("write your pallas kernel here")


# speedup vs baseline: 5.1489x; 5.1489x over previous
import functools

import jax
import jax.numpy as jnp
from jax import lax
from jax.experimental import pallas as pl
from jax.experimental.pallas import tpu as pltpu
from jax.experimental.pallas import tpu_sc as plsc

_N1, _N2, _N3 = 10000, 2500, 625
_E1, _E2 = 320000, 80000
_EU1, _EU2 = 20000, 80000
_CIN, _COUT = 128, 128
_K3, _K2 = 27, 8

_F32 = jnp.float32
_I32 = jnp.int32

_CH = 128
_ZB = 128

_mesh = plsc.VectorSubcoreMesh(core_axis_name="c", subcore_axis_name="s")
_sc_params = pltpu.CompilerParams(use_tc_tiling_on_sc=False)


def _cdiv(a, b):
    return (a + b - 1) // b


def _make_sc_conv(E, n_seg, A, B):
    assert E % _CH == 0 and n_seg % _ZB == 0
    nch = E // _CH
    nchw = _cdiv(nch, 32)
    nz = n_seg // _ZB
    nzw = _cdiv(nz, 16)

    @functools.partial(
        pl.kernel,
        out_type=jax.ShapeDtypeStruct((2, n_seg, 16), _F32),
        mesh=_mesh,
        compiler_params=_sc_params,
        scratch_types=[
            pltpu.VMEM((_CH,), _I32),
            pltpu.VMEM((_CH,), _I32),
            pltpu.VMEM((_CH,), _I32),
            pltpu.VMEM((_CH,), _I32),
            pltpu.VMEM((_CH,), _I32),
            pltpu.VMEM((_CH, 16), _F32),
            pltpu.VMEM((_ZB, 16), _F32),
            pltpu.VMEM_SHARED((n_seg, 16), _F32),
            pltpu.SemaphoreType.DMA,
        ],
    )
    def conv(table, src, dst, kid, out, src_v, dst_v, kid_v, gi_v, si_v,
             rows_v, zero_v, acc, sem):
        c = lax.axis_index("c")
        s = lax.axis_index("s")
        w = s * 2 + c

        @pl.loop(0, _ZB)
        def _zb(i):
            zero_v[i] = jnp.zeros((16,), _F32)

        @pl.loop(0, nzw)
        def _zero(j):
            z = s + j * 16

            @pl.when(z < nz)
            def _():
                off = pl.multiple_of(z * _ZB, _ZB)
                pltpu.sync_copy(zero_v, acc.at[pl.ds(off, _ZB)])

        plsc.subcore_barrier()

        @pl.loop(0, nchw)
        def _edges(it):
            cid = w + it * 32

            @pl.when(cid < nch)
            def _():
                base = pl.multiple_of(cid * _CH, _CH)
                pltpu.sync_copy(src.at[pl.ds(base, _CH)], src_v)
                pltpu.sync_copy(dst.at[pl.ds(base, _CH)], dst_v)
                if A > 1 or B > 1:
                    pltpu.sync_copy(kid.at[pl.ds(base, _CH)], kid_v)

                if A > 1:
                    @pl.loop(0, _CH // 16)
                    def _gi(i):
                        sl = pl.ds(i * 16, 16)
                        gi_v[sl] = src_v[sl] * A + kid_v[sl]
                    gidx = gi_v
                else:
                    gidx = src_v

                if B > 1:
                    @pl.loop(0, _CH // 16)
                    def _si(i):
                        sl = pl.ds(i * 16, 16)
                        si_v[sl] = dst_v[sl] * B + kid_v[sl]
                    sidx = si_v
                else:
                    sidx = dst_v

                pltpu.async_copy(table.at[gidx], rows_v, sem).wait()
                pltpu.sync_copy(rows_v, acc.at[sidx], add=True)

        plsc.subcore_barrier()

        @pl.loop(0, nzw)
        def _dump(j):
            z = s + j * 16

            @pl.when(z < nz)
            def _():
                sl = pl.ds(pl.multiple_of(z * _ZB, _ZB), _ZB)
                pltpu.sync_copy(acc.at[sl], out.at[c].at[sl])

    return conv


def _make_sc_pool(n_child, n_par):
    assert n_child % _CH == 0
    nch = n_child // _CH
    nchw = _cdiv(nch, 32)

    @functools.partial(
        pl.kernel,
        out_type=jax.ShapeDtypeStruct((32, n_par, 16), _F32),
        mesh=_mesh,
        compiler_params=_sc_params,
        scratch_types=[
            pltpu.VMEM((_CH, 16), _F32),
            pltpu.VMEM((_CH, 16), _F32),
            pltpu.VMEM((_CH,), _I32),
            pltpu.VMEM((n_par, 16), _F32),
        ],
    )
    def pool(parts, parent, out, a_v, b_v, par_v, pacc):
        c = lax.axis_index("c")
        s = lax.axis_index("s")
        w = s * 2 + c

        @pl.loop(0, n_par)
        def _zero(i):
            pacc[i] = jnp.zeros((16,), _F32)

        @pl.loop(0, nchw)
        def _chunks(it):
            cid = w + it * 32

            @pl.when(cid < nch)
            def _():
                base = pl.multiple_of(cid * _CH, _CH)
                pltpu.sync_copy(parts.at[0].at[pl.ds(base, _CH)], a_v)
                pltpu.sync_copy(parts.at[1].at[pl.ds(base, _CH)], b_v)
                pltpu.sync_copy(parent.at[pl.ds(base, _CH)], par_v)

                @pl.loop(0, _CH // 16)
                def _grp(g):
                    pvec = par_v[pl.ds(g * 16, 16)]
                    for j in range(16):
                        p = pvec[j]
                        i = g * 16 + j
                        v = a_v[i] + b_v[i]
                        pacc[p] = jnp.maximum(pacc[p], v)

        pltpu.sync_copy(pacc, out.at[w])

    return pool


def _tc_mm(x, w, n_out):
    def body(x_ref, w_ref, o_ref):
        o_ref[...] = jnp.dot(x_ref[...], w_ref[...],
                             preferred_element_type=_F32)

    return pl.pallas_call(
        body,
        out_shape=jax.ShapeDtypeStruct((x.shape[0], n_out), _F32),
    )(x, w)


def _tc_max_mm(q, w, n_out):
    def body(q_ref, w_ref, o_ref):
        m = jnp.max(q_ref[...], axis=0)
        o_ref[...] = jnp.dot(m, w_ref[...], preferred_element_type=_F32)

    return pl.pallas_call(
        body,
        out_shape=jax.ShapeDtypeStruct((q.shape[1], n_out), _F32),
    )(q, w)


def _tc_relu_sum(p):
    def body(p_ref, o_ref):
        o_ref[...] = jnp.maximum(p_ref[0] + p_ref[1], 0.0)

    return pl.pallas_call(
        body,
        out_shape=jax.ShapeDtypeStruct(p.shape[1:], _F32),
    )(p)


def _tc_final(a, b, w):
    def body(a_ref, b_ref, w_ref, o_ref):
        z = jnp.dot(a_ref[...] + b_ref[...], w_ref[...],
                    preferred_element_type=_F32)
        o_ref[...] = jax.nn.sigmoid(z)

    return pl.pallas_call(
        body,
        out_shape=jax.ShapeDtypeStruct((a.shape[0], w.shape[1]), _F32),
    )(a, b, w)


_N1P = 10240
_N2P = 2560
_EU1P = 20096

_conv1 = _make_sc_conv(_E1, _N1P, _K3, 1)
_conv2 = _make_sc_conv(_E2, _N2P, _K3, 1)
_conv3 = _make_sc_conv(_EU1P, _N2P, _K2, 1)
_conv4 = _make_sc_conv(_EU2, _N1 * _K2, 1, _K2)
_pool1 = _make_sc_pool(_N1P, _N2)
_pool2 = _make_sc_pool(_N2P, _N3)


def kernel(x, edge_index1, kidx1, parent1, edge_index2, kidx2, parent2,
           src_u1, dst_u1, kidx_u1, src_u2, dst_u2, kidx_u2,
           W1, W2, Wt1, Wt2):
    W1r = jnp.transpose(W1, (1, 0, 2)).reshape(_CIN, _K3 * 16)
    T1 = _tc_mm(x, W1r, _K3 * 16).reshape(_N1 * _K3, 16)
    P1 = _conv1(T1, edge_index1[0], edge_index1[1], kidx1)
    par1p = jnp.pad(parent1, (0, _N1P - _N1))
    Q1 = _pool1(P1, par1p)

    W2p = jnp.pad(W2, ((0, 0), (0, 0), (0, 12)))
    W2r = jnp.transpose(W2p, (1, 0, 2)).reshape(16, _K3 * 16)
    T2 = _tc_max_mm(Q1, W2r, _K3 * 16).reshape(_N2 * _K3, 16)
    P2 = _conv2(T2, edge_index2[0], edge_index2[1], kidx2)
    par2p = jnp.pad(parent2, (0, _N2P - _N2))
    Q2 = _pool2(P2, par2p)

    Wt1p = jnp.pad(Wt1, ((0, 0), (0, 12), (0, 0)))
    Wt1r = jnp.transpose(Wt1p, (1, 0, 2)).reshape(16, _K2 * 16)
    T3 = _tc_max_mm(Q2, Wt1r, _K2 * 16).reshape(_N3 * _K2, 16)
    npad = _EU1P - _EU1
    src1p = jnp.pad(src_u1, (0, npad))
    dst1p = jnp.pad(dst_u1, (0, npad), constant_values=_N2)
    kid1p = jnp.pad(kidx_u1, (0, npad))
    P3 = _conv3(T3, src1p, dst1p, kid1p)
    H3 = _tc_relu_sum(P3)

    P4 = _conv4(H3, src_u2, dst_u2, kidx_u2)
    A4 = P4.reshape(2, _N1, _K2 * 16)
    Wt2r = Wt2.reshape(_K2 * 16, _COUT)
    return _tc_final(A4[0], A4[1], Wt2r)


# speedup vs baseline: 8.4752x; 1.6460x over previous
import functools

import jax
import jax.numpy as jnp
from jax import lax
from jax.experimental import pallas as pl
from jax.experimental.pallas import tpu as pltpu
from jax.experimental.pallas import tpu_sc as plsc

_N1, _N2, _N3 = 10000, 2500, 625
_E1, _E2 = 320000, 80000
_EU1, _EU2 = 20000, 80000
_CIN, _COUT = 128, 128
_K3, _K2 = 27, 8

_F32 = jnp.float32
_I32 = jnp.int32

_CH = 128
_ZB = 128

_mesh = plsc.VectorSubcoreMesh(core_axis_name="c", subcore_axis_name="s")
_sc_params = pltpu.CompilerParams(use_tc_tiling_on_sc=False)


def _cdiv(a, b):
    return (a + b - 1) // b


def _make_sc_conv(E, n_seg, A, B):
    assert E % _CH == 0 and n_seg % _ZB == 0
    nch = E // _CH
    nchw = _cdiv(nch, 32)
    npair = (nchw + 2) // 2
    nz = n_seg // _ZB
    nzw = _cdiv(nz, 16)
    nzg = _cdiv(nzw, 8)

    @functools.partial(
        pl.kernel,
        out_type=jax.ShapeDtypeStruct((2, n_seg, 16), _F32),
        mesh=_mesh,
        compiler_params=_sc_params,
        scratch_types=[
            pltpu.VMEM((2, 3, _CH), _I32),
            pltpu.VMEM((2, _CH), _I32),
            pltpu.VMEM((2, _CH), _I32),
            pltpu.VMEM((2, _CH, 16), _F32),
            pltpu.VMEM((_ZB, 16), _F32),
            pltpu.VMEM_SHARED((n_seg, 16), _F32),
            pltpu.SemaphoreType.DMA((2,)),
            pltpu.SemaphoreType.DMA((2,)),
            pltpu.SemaphoreType.DMA,
        ],
    )
    def conv(table, e3, out, idx3_v, gi_v, si_v, rows_v, zero_v, acc,
             sem_i, sem_g, sem_z):
        c = lax.axis_index("c")
        s = lax.axis_index("s")
        w = s * 2 + c

        @pl.loop(0, _ZB)
        def _zb(i):
            zero_v[i] = jnp.zeros((16,), _F32)

        @pl.loop(0, nzg)
        def _zero(jo):
            for g in range(8):
                z = s + (jo * 8 + g) * 16

                @pl.when(z < nz)
                def _():
                    off = pl.multiple_of(z * _ZB, _ZB)
                    pltpu.async_copy(zero_v, acc.at[pl.ds(off, _ZB)], sem_z)
            for g in range(8):
                z = s + (jo * 8 + g) * 16

                @pl.when(z < nz)
                def _():
                    off = pl.multiple_of(z * _ZB, _ZB)
                    pltpu.make_async_copy(
                        zero_v, acc.at[pl.ds(off, _ZB)], sem_z).wait()

        plsc.subcore_barrier()

        def guard(j, fn):
            cid = w + j * 32
            pl.when(jnp.logical_and(cid >= 0, cid < nch))(fn(cid))

        def issue_idx(j, b):
            def f(cid):
                def body():
                    base = pl.multiple_of(cid * _CH, _CH)
                    pltpu.async_copy(e3.at[:, pl.ds(base, _CH)],
                                     idx3_v.at[b], sem_i.at[b])
                return body
            guard(j, f)

        def wait_idx(j, b):
            def f(cid):
                def body():
                    pltpu.make_async_copy(e3.at[:, pl.ds(0, _CH)],
                                          idx3_v.at[b], sem_i.at[b]).wait()
                return body
            guard(j, f)

        def compute_idx(j, b):
            def f(cid):
                def body():
                    @pl.loop(0, _CH // 16)
                    def _(i):
                        sl = pl.ds(i * 16, 16)
                        if A > 1:
                            gi_v[b, sl] = idx3_v[b, 0, sl] * A + idx3_v[b, 2, sl]
                        else:
                            gi_v[b, sl] = idx3_v[b, 0, sl]
                        if B > 1:
                            si_v[b, sl] = idx3_v[b, 1, sl] * B + idx3_v[b, 2, sl]
                        else:
                            si_v[b, sl] = idx3_v[b, 1, sl]
                return body
            guard(j, f)

        def issue_gather(j, b):
            def f(cid):
                def body():
                    pltpu.async_copy(table.at[gi_v.at[b]], rows_v.at[b],
                                     sem_g.at[b])
                return body
            guard(j, f)

        def wait_gather_scatter(j, b):
            def f(cid):
                def body():
                    pltpu.make_async_copy(table.at[gi_v.at[b]], rows_v.at[b],
                                          sem_g.at[b]).wait()
                    pltpu.sync_copy(rows_v.at[b], acc.at[si_v.at[b]], add=True)
                return body
            guard(j, f)

        issue_idx(0, 0)
        issue_idx(1, 1)

        @pl.loop(0, npair)
        def _pipe(jj):
            for b in (0, 1):
                j = jj * 2 + b
                wait_idx(j, b)
                compute_idx(j, b)
                issue_idx(j + 2, b)
                issue_gather(j, b)
                wait_gather_scatter(j - 1, 1 - b)

        plsc.subcore_barrier()

        @pl.loop(0, nzg)
        def _dump(jo):
            for g in range(8):
                z = s + (jo * 8 + g) * 16

                @pl.when(z < nz)
                def _():
                    sl = pl.ds(pl.multiple_of(z * _ZB, _ZB), _ZB)
                    pltpu.async_copy(acc.at[sl], out.at[c].at[sl], sem_z)
            for g in range(8):
                z = s + (jo * 8 + g) * 16

                @pl.when(z < nz)
                def _():
                    sl = pl.ds(pl.multiple_of(z * _ZB, _ZB), _ZB)
                    pltpu.make_async_copy(acc.at[sl], out.at[c].at[sl],
                                          sem_z).wait()

    return conv


def _make_sc_pool(n_child, n_par):
    assert n_child % _CH == 0
    nch = n_child // _CH
    nchw = _cdiv(nch, 32)

    @functools.partial(
        pl.kernel,
        out_type=jax.ShapeDtypeStruct((32, n_par, 16), _F32),
        mesh=_mesh,
        compiler_params=_sc_params,
        scratch_types=[
            pltpu.VMEM((_CH, 16), _F32),
            pltpu.VMEM((_CH, 16), _F32),
            pltpu.VMEM((_CH,), _I32),
            pltpu.VMEM((n_par, 16), _F32),
        ],
    )
    def pool(parts, parent, out, a_v, b_v, par_v, pacc):
        c = lax.axis_index("c")
        s = lax.axis_index("s")
        w = s * 2 + c

        @pl.loop(0, n_par)
        def _zero(i):
            pacc[i] = jnp.zeros((16,), _F32)

        @pl.loop(0, nchw)
        def _chunks(it):
            cid = w + it * 32

            @pl.when(cid < nch)
            def _():
                base = pl.multiple_of(cid * _CH, _CH)
                pltpu.sync_copy(parts.at[0].at[pl.ds(base, _CH)], a_v)
                pltpu.sync_copy(parts.at[1].at[pl.ds(base, _CH)], b_v)
                pltpu.sync_copy(parent.at[pl.ds(base, _CH)], par_v)

                @pl.loop(0, _CH // 16)
                def _grp(g):
                    pvec = par_v[pl.ds(g * 16, 16)]
                    for j in range(16):
                        p = pvec[j]
                        i = g * 16 + j
                        v = a_v[i] + b_v[i]
                        pacc[p] = jnp.maximum(pacc[p], v)

        pltpu.sync_copy(pacc, out.at[w])

    return pool


def _tc_mm(x, w, n_out):
    def body(x_ref, w_ref, o_ref):
        o_ref[...] = jnp.dot(x_ref[...], w_ref[...],
                             preferred_element_type=_F32)

    return pl.pallas_call(
        body,
        out_shape=jax.ShapeDtypeStruct((x.shape[0], n_out), _F32),
    )(x, w)


def _tc_max_mm(q, w, n_out):
    def body(q_ref, w_ref, o_ref):
        m = jnp.max(q_ref[...], axis=0)
        o_ref[...] = jnp.dot(m, w_ref[...], preferred_element_type=_F32)

    return pl.pallas_call(
        body,
        out_shape=jax.ShapeDtypeStruct((q.shape[1], n_out), _F32),
    )(q, w)


def _tc_relu_sum(p):
    def body(p_ref, o_ref):
        o_ref[...] = jnp.maximum(p_ref[0] + p_ref[1], 0.0)

    return pl.pallas_call(
        body,
        out_shape=jax.ShapeDtypeStruct(p.shape[1:], _F32),
    )(p)


def _tc_final(a, b, w):
    def body(a_ref, b_ref, w_ref, o_ref):
        z = jnp.dot(a_ref[...] + b_ref[...], w_ref[...],
                    preferred_element_type=_F32)
        o_ref[...] = jax.nn.sigmoid(z)

    return pl.pallas_call(
        body,
        out_shape=jax.ShapeDtypeStruct((a.shape[0], w.shape[1]), _F32),
    )(a, b, w)


_N1P = 10240
_N2P = 2560
_EU1P = 20096

_conv1 = _make_sc_conv(_E1, _N1P, _K3, 1)
_conv2 = _make_sc_conv(_E2, _N2P, _K3, 1)
_conv3 = _make_sc_conv(_EU1P, _N2P, _K2, 1)
_conv4 = _make_sc_conv(_EU2, _N1 * _K2, 1, _K2)
_pool1 = _make_sc_pool(_N1P, _N2)
_pool2 = _make_sc_pool(_N2P, _N3)


def kernel(x, edge_index1, kidx1, parent1, edge_index2, kidx2, parent2,
           src_u1, dst_u1, kidx_u1, src_u2, dst_u2, kidx_u2,
           W1, W2, Wt1, Wt2):
    W1r = jnp.transpose(W1, (1, 0, 2)).reshape(_CIN, _K3 * 16)
    T1 = _tc_mm(x, W1r, _K3 * 16).reshape(_N1 * _K3, 16)
    e1 = jnp.concatenate([edge_index1, kidx1[None]], axis=0)
    P1 = _conv1(T1, e1)
    par1p = jnp.pad(parent1, (0, _N1P - _N1))
    Q1 = _pool1(P1, par1p)

    W2p = jnp.pad(W2, ((0, 0), (0, 0), (0, 12)))
    W2r = jnp.transpose(W2p, (1, 0, 2)).reshape(16, _K3 * 16)
    T2 = _tc_max_mm(Q1, W2r, _K3 * 16).reshape(_N2 * _K3, 16)
    e2 = jnp.concatenate([edge_index2, kidx2[None]], axis=0)
    P2 = _conv2(T2, e2)
    par2p = jnp.pad(parent2, (0, _N2P - _N2))
    Q2 = _pool2(P2, par2p)

    Wt1p = jnp.pad(Wt1, ((0, 0), (0, 12), (0, 0)))
    Wt1r = jnp.transpose(Wt1p, (1, 0, 2)).reshape(16, _K2 * 16)
    T3 = _tc_max_mm(Q2, Wt1r, _K2 * 16).reshape(_N3 * _K2, 16)
    npad = _EU1P - _EU1
    src1p = jnp.pad(src_u1, (0, npad))
    dst1p = jnp.pad(dst_u1, (0, npad), constant_values=_N2)
    kid1p = jnp.pad(kidx_u1, (0, npad))
    eu1 = jnp.stack([src1p, dst1p, kid1p])
    P3 = _conv3(T3, eu1)
    H3 = _tc_relu_sum(P3)

    eu2 = jnp.stack([src_u2, dst_u2, kidx_u2])
    P4 = _conv4(H3, eu2)
    A4 = P4.reshape(2, _N1, _K2 * 16)
    Wt2r = Wt2.reshape(_K2 * 16, _COUT)
    return _tc_final(A4[0], A4[1], Wt2r)


# speedup vs baseline: 13.7685x; 1.6246x over previous
import functools

import jax
import jax.numpy as jnp
from jax import lax
from jax.experimental import pallas as pl
from jax.experimental.pallas import tpu as pltpu
from jax.experimental.pallas import tpu_sc as plsc

_N1, _N2, _N3 = 10000, 2500, 625
_E1, _E2 = 320000, 80000
_EU1, _EU2 = 20000, 80000
_CIN, _COUT = 128, 128
_K3, _K2 = 27, 8

_F32 = jnp.float32
_I32 = jnp.int32

_CH = 128
_ZB = 128

_N1P = 10240
_N2P = 2560
_N3P = 640
_EU1P = 20096

_mesh = plsc.VectorSubcoreMesh(core_axis_name="c", subcore_axis_name="s")
_sc_params = pltpu.CompilerParams(use_tc_tiling_on_sc=False)


def _cdiv(a, b):
    return (a + b - 1) // b


def _make_sc_conv(E, n_seg, SLOT, B, partition=False):
    assert E % _CH == 0 and n_seg % _ZB == 0
    nch = E // _CH
    nw = 16 if partition else 32
    nchw = _cdiv(nch, nw)
    npair = (nchw + 2) // 2
    half = n_seg // 2
    if partition:
        assert half % _ZB == 0
    nz = (half if partition else n_seg) // _ZB
    nzz = n_seg // _ZB
    nzw = _cdiv(nz, 16)
    nzzw = _cdiv(nzz, 16)
    nzg = _cdiv(nzzw, 8)
    out_shape = (n_seg, 16) if partition else (2, n_seg, 16)

    @functools.partial(
        pl.kernel,
        out_type=jax.ShapeDtypeStruct(out_shape, _F32),
        mesh=_mesh,
        compiler_params=_sc_params,
        scratch_types=[
            pltpu.VMEM((2, 3, _CH), _I32),
            pltpu.VMEM((2, _CH), _I32),
            pltpu.VMEM((2, _CH), _I32),
            pltpu.VMEM((2, _CH, 16), _F32),
            pltpu.VMEM((_ZB, 16), _F32),
            pltpu.VMEM_SHARED((n_seg, 16), _F32),
            pltpu.SemaphoreType.DMA((2,)),
            pltpu.SemaphoreType.DMA((2,)),
            pltpu.SemaphoreType.DMA,
        ],
    )
    def conv(table, e3, out, idx3_v, gi_v, si_v, rows_v, zero_v, acc,
             sem_i, sem_g, sem_z):
        c = lax.axis_index("c")
        s = lax.axis_index("s")
        w = (s * 2 + c) if not partition else s

        @pl.loop(0, _ZB)
        def _zb(i):
            zero_v[i] = jnp.zeros((16,), _F32)

        @pl.loop(0, nzg)
        def _zero(jo):
            for g in range(8):
                z = s + (jo * 8 + g) * 16

                @pl.when(z < nzz)
                def _():
                    off = pl.multiple_of(z * _ZB, _ZB)
                    pltpu.async_copy(zero_v, acc.at[pl.ds(off, _ZB)], sem_z)
            for g in range(8):
                z = s + (jo * 8 + g) * 16

                @pl.when(z < nzz)
                def _():
                    off = pl.multiple_of(z * _ZB, _ZB)
                    pltpu.make_async_copy(
                        zero_v, acc.at[pl.ds(off, _ZB)], sem_z).wait()

        plsc.subcore_barrier()

        def guard(j, fn):
            cid = w + j * nw
            pl.when(jnp.logical_and(cid >= 0, cid < nch))(fn(cid))

        def issue_idx(j, b):
            def f(cid):
                def body():
                    base = pl.multiple_of(cid * _CH, _CH)
                    for r in range(3):
                        pltpu.async_copy(e3.at[pl.ds(r * E + base, _CH)],
                                         idx3_v.at[b, r], sem_i.at[b])
                return body
            guard(j, f)

        def wait_idx(j, b):
            def f(cid):
                def body():
                    for r in range(3):
                        pltpu.make_async_copy(e3.at[pl.ds(r * E, _CH)],
                                              idx3_v.at[b, r],
                                              sem_i.at[b]).wait()
                return body
            guard(j, f)

        def compute_idx(j, b):
            def f(cid):
                def body():
                    @pl.loop(0, _CH // 16)
                    def _(i):
                        sl = pl.ds(i * 16, 16)
                        if SLOT > 1:
                            gi_v[b, sl] = (idx3_v[b, 0, sl] * SLOT
                                           + idx3_v[b, 2, sl])
                        else:
                            gi_v[b, sl] = idx3_v[b, 0, sl]
                        if B > 1:
                            si = idx3_v[b, 1, sl] * B + idx3_v[b, 2, sl]
                        else:
                            si = idx3_v[b, 1, sl]
                        if partition:
                            lo = c * half
                            owned = jnp.logical_and(si >= lo, si < lo + half)
                            trash = (1 - c) * (n_seg - 1)
                            si = jnp.where(owned, si, trash)
                        si_v[b, sl] = si
                return body
            guard(j, f)

        def issue_gather(j, b):
            def f(cid):
                def body():
                    pltpu.async_copy(table.at[gi_v.at[b]], rows_v.at[b],
                                     sem_g.at[b])
                return body
            guard(j, f)

        def wait_gather_scatter(j, b):
            def f(cid):
                def body():
                    pltpu.make_async_copy(table.at[gi_v.at[b]], rows_v.at[b],
                                          sem_g.at[b]).wait()
                    pltpu.sync_copy(rows_v.at[b], acc.at[si_v.at[b]], add=True)
                return body
            guard(j, f)

        issue_idx(0, 0)
        issue_idx(1, 1)

        @pl.loop(0, npair)
        def _pipe(jj):
            for b in (0, 1):
                j = jj * 2 + b
                wait_idx(j, b)
                compute_idx(j, b)
                issue_idx(j + 2, b)
                issue_gather(j, b)
                wait_gather_scatter(j - 1, 1 - b)

        plsc.subcore_barrier()

        if partition:
            @pl.loop(0, nzw)
            def _dump(j):
                z = s + j * 16

                @pl.when(z < nz)
                def _():
                    off = pl.multiple_of(c * half + z * _ZB, _ZB)
                    sl = pl.ds(off, _ZB)
                    pltpu.sync_copy(acc.at[sl], zero_v)

                    @pl.loop(0, _ZB)
                    def _relu(i):
                        zero_v[i] = jnp.maximum(zero_v[i], 0.0)

                    pltpu.sync_copy(zero_v, out.at[sl])
        else:
            @pl.loop(0, nzg)
            def _dump(jo):
                for g in range(8):
                    z = s + (jo * 8 + g) * 16

                    @pl.when(z < nz)
                    def _():
                        sl = pl.ds(pl.multiple_of(z * _ZB, _ZB), _ZB)
                        pltpu.async_copy(acc.at[sl], out.at[c].at[sl], sem_z)
                for g in range(8):
                    z = s + (jo * 8 + g) * 16

                    @pl.when(z < nz)
                    def _():
                        sl = pl.ds(pl.multiple_of(z * _ZB, _ZB), _ZB)
                        pltpu.make_async_copy(acc.at[sl], out.at[c].at[sl],
                                              sem_z).wait()

    return conv


def _make_sc_pool(n_child, n_par):
    assert n_child % _CH == 0 and n_par % _ZB == 0
    nch = n_child // _CH
    nchw = _cdiv(nch, 32)
    nr = n_par // _ZB
    nrw = _cdiv(nr, 16)

    @functools.partial(
        pl.kernel,
        out_type=jax.ShapeDtypeStruct((32, n_par, 16), _F32),
        mesh=_mesh,
        compiler_params=_sc_params,
        scratch_types=[
            pltpu.VMEM((_CH, 16), _F32),
            pltpu.VMEM((_CH, 16), _F32),
            pltpu.VMEM((_CH,), _I32),
            pltpu.VMEM((n_par, 16), _F32),
            pltpu.VMEM((_ZB, 16), _F32),
            pltpu.VMEM((_ZB, 16), _F32),
            pltpu.VMEM_SHARED((16, n_par, 16), _F32),
        ],
    )
    def pool(parts, parent, out, a_v, b_v, par_v, pacc, red_v, tmp_v, stage):
        c = lax.axis_index("c")
        s = lax.axis_index("s")
        w = s * 2 + c

        @pl.loop(0, n_par // 4)
        def _zero(i):
            for r in range(4):
                pacc[i * 4 + r] = jnp.zeros((16,), _F32)

        @pl.loop(0, nchw)
        def _chunks(it):
            cid = w + it * 32

            @pl.when(cid < nch)
            def _():
                base = pl.multiple_of(cid * _CH, _CH)
                pltpu.sync_copy(parts.at[0].at[pl.ds(base, _CH)], a_v)
                pltpu.sync_copy(parts.at[1].at[pl.ds(base, _CH)], b_v)
                pltpu.sync_copy(parent.at[pl.ds(base, _CH)], par_v)

                @pl.loop(0, _CH // 16)
                def _grp(g):
                    pvec = par_v[pl.ds(g * 16, 16)]
                    for j in range(16):
                        p = pvec[j]
                        i = g * 16 + j
                        v = a_v[i] + b_v[i]
                        pacc[p] = jnp.maximum(pacc[p], v)

        pltpu.sync_copy(pacc, out.at[w])

    return pool


def _tc_mm(x, w, n_out):
    def body(x_ref, w_ref, o_ref):
        o_ref[...] = jnp.dot(x_ref[...], w_ref[...],
                             preferred_element_type=_F32)

    return pl.pallas_call(
        body,
        out_shape=jax.ShapeDtypeStruct((x.shape[0], n_out), _F32),
    )(x, w)


def _tc_max_mm(q, w, n_out):
    def body(q_ref, w_ref, o_ref):
        m = jnp.max(q_ref[...], axis=0)
        o_ref[...] = jnp.dot(m, w_ref[...], preferred_element_type=_F32)

    return pl.pallas_call(
        body,
        out_shape=jax.ShapeDtypeStruct((q.shape[1], n_out), _F32),
    )(q, w)


def _tc_final(p, w):
    def body(p_ref, w_ref, o_ref):
        z = jnp.dot(p_ref[0] + p_ref[1], w_ref[...],
                    preferred_element_type=_F32)
        o_ref[...] = jax.nn.sigmoid(z)

    return pl.pallas_call(
        body,
        out_shape=jax.ShapeDtypeStruct((p.shape[1], w.shape[1]), _F32),
    )(p, w)


_conv1 = _make_sc_conv(_E1, _N1P, 32, 1)
_conv2 = _make_sc_conv(_E2, _N2P, 32, 1)
_conv3 = _make_sc_conv(_EU1P, _N2P, _K2, 1)
_conv4 = _make_sc_conv(_EU2, _N1 * _K2, 1, _K2)
_pool1 = _make_sc_pool(_N1P, _N2P)
_pool2 = _make_sc_pool(_N2P, _N3P)


def _slot_pad(wr, k, cout):
    cin = wr.shape[0]
    w3 = wr.reshape(cin, k, cout)
    w3 = jnp.pad(w3, ((0, 0), (0, 32 - k), (0, 16 - cout)))
    return w3.reshape(cin, 512)


def kernel(x, edge_index1, kidx1, parent1, edge_index2, kidx2, parent2,
           src_u1, dst_u1, kidx_u1, src_u2, dst_u2, kidx_u2,
           W1, W2, Wt1, Wt2):
    W1r = _slot_pad(jnp.transpose(W1, (1, 0, 2)).reshape(_CIN, _K3 * 16),
                    _K3, 16)
    T1 = _tc_mm(x, W1r, 512).reshape(_N1 * 32, 16)
    e1 = jnp.concatenate([edge_index1[0], edge_index1[1], kidx1])
    P1 = _conv1(T1, e1)
    par1p = jnp.pad(parent1, (0, _N1P - _N1))
    Q1 = _pool1(P1, par1p)

    W2r = _slot_pad(jnp.transpose(W2, (1, 0, 2)).reshape(16, _K3 * 4), _K3, 4)
    T2 = _tc_max_mm(Q1, W2r, 512).reshape(_N2P * 32, 16)
    e2 = jnp.concatenate([edge_index2[0], edge_index2[1], kidx2])
    P2 = _conv2(T2, e2)
    par2p = jnp.pad(parent2, (0, _N2P - _N2))
    Q2 = _pool2(P2, par2p)

    Wt1p = jnp.pad(Wt1, ((0, 0), (0, 12), (0, 0)))
    Wt1r = jnp.transpose(Wt1p, (1, 0, 2)).reshape(16, _K2 * 16)
    T3 = _tc_max_mm(Q2, Wt1r, _K2 * 16).reshape(_N3P * _K2, 16)
    npad = _EU1P - _EU1
    eu1 = jnp.concatenate([
        jnp.pad(src_u1, (0, npad)),
        jnp.pad(dst_u1, (0, npad), constant_values=_N2),
        jnp.pad(kidx_u1, (0, npad)),
    ])
    P3 = _conv3(T3, eu1)

    def _relu_body(p_ref, o_ref):
        o_ref[...] = jnp.maximum(p_ref[0] + p_ref[1], 0.0)

    H3 = pl.pallas_call(
        _relu_body,
        out_shape=jax.ShapeDtypeStruct((_N2P, 16), _F32),
    )(P3)

    eu2 = jnp.concatenate([src_u2, dst_u2, kidx_u2])
    P4 = _conv4(H3, eu2)
    A4 = P4.reshape(2, _N1, _K2 * 16)
    Wt2r = Wt2.reshape(_K2 * 16, _COUT)
    return _tc_final(A4, Wt2r)


# speedup vs baseline: 15.5867x; 1.1321x over previous
import functools

import jax
import jax.numpy as jnp
from jax import lax
from jax.experimental import pallas as pl
from jax.experimental.pallas import tpu as pltpu
from jax.experimental.pallas import tpu_sc as plsc

_N1, _N2, _N3 = 10000, 2500, 625
_E1, _E2 = 320000, 80000
_EU1, _EU2 = 20000, 80000
_CIN, _COUT = 128, 128
_K3, _K2 = 27, 8

_F32 = jnp.float32
_I32 = jnp.int32

_CH = 128
_ZB = 128

_N1P = 10240
_N2P = 2560
_N3P = 640
_EU1P = 20096

_mesh = plsc.VectorSubcoreMesh(core_axis_name="c", subcore_axis_name="s")
_sc_params = pltpu.CompilerParams(use_tc_tiling_on_sc=False)


def _cdiv(a, b):
    return (a + b - 1) // b


def _make_sc_conv(E, n_seg, SLOT, B, partition=False):
    assert E % _CH == 0 and n_seg % _ZB == 0
    nch = E // _CH
    nw = 16 if partition else 32
    nchw = _cdiv(nch, nw)
    npair = (nchw + 2) // 2
    half = n_seg // 2
    if partition:
        assert half % _ZB == 0
    nz = (half if partition else n_seg) // _ZB
    nzz = n_seg // _ZB
    nzw = _cdiv(nz, 16)
    nzzw = _cdiv(nzz, 16)
    nzg = _cdiv(nzzw, 8)
    out_shape = (n_seg, 16) if partition else (2, n_seg, 16)

    @functools.partial(
        pl.kernel,
        out_type=jax.ShapeDtypeStruct(out_shape, _F32),
        mesh=_mesh,
        compiler_params=_sc_params,
        scratch_types=[
            pltpu.VMEM((2, 3, _CH), _I32),
            pltpu.VMEM((2, _CH), _I32),
            pltpu.VMEM((2, _CH), _I32),
            pltpu.VMEM((2, _CH, 16), _F32),
            pltpu.VMEM((_ZB, 16), _F32),
            pltpu.VMEM_SHARED((n_seg, 16), _F32),
            pltpu.SemaphoreType.DMA((2,)),
            pltpu.SemaphoreType.DMA((2,)),
            pltpu.SemaphoreType.DMA,
        ],
    )
    def conv(table, e3, out, idx3_v, gi_v, si_v, rows_v, zero_v, acc,
             sem_i, sem_g, sem_z):
        c = lax.axis_index("c")
        s = lax.axis_index("s")
        w = (s * 2 + c) if not partition else s

        @pl.loop(0, _ZB)
        def _zb(i):
            zero_v[i] = jnp.zeros((16,), _F32)

        @pl.loop(0, nzg)
        def _zero(jo):
            for g in range(8):
                z = s + (jo * 8 + g) * 16

                @pl.when(z < nzz)
                def _():
                    off = pl.multiple_of(z * _ZB, _ZB)
                    pltpu.async_copy(zero_v, acc.at[pl.ds(off, _ZB)], sem_z)
            for g in range(8):
                z = s + (jo * 8 + g) * 16

                @pl.when(z < nzz)
                def _():
                    off = pl.multiple_of(z * _ZB, _ZB)
                    pltpu.make_async_copy(
                        zero_v, acc.at[pl.ds(off, _ZB)], sem_z).wait()

        plsc.subcore_barrier()

        def guard(j, fn):
            cid = w + j * nw
            pl.when(jnp.logical_and(cid >= 0, cid < nch))(fn(cid))

        def issue_idx(j, b):
            def f(cid):
                def body():
                    base = pl.multiple_of(cid * _CH, _CH)
                    for r in range(3):
                        pltpu.async_copy(e3.at[pl.ds(r * E + base, _CH)],
                                         idx3_v.at[b, r], sem_i.at[b])
                return body
            guard(j, f)

        def wait_idx(j, b):
            def f(cid):
                def body():
                    for r in range(3):
                        pltpu.make_async_copy(e3.at[pl.ds(r * E, _CH)],
                                              idx3_v.at[b, r],
                                              sem_i.at[b]).wait()
                return body
            guard(j, f)

        def compute_idx(j, b):
            def f(cid):
                def body():
                    @pl.loop(0, _CH // 16)
                    def _(i):
                        sl = pl.ds(i * 16, 16)
                        if SLOT > 1:
                            gi_v[b, sl] = (idx3_v[b, 0, sl] * SLOT
                                           + idx3_v[b, 2, sl])
                        else:
                            gi_v[b, sl] = idx3_v[b, 0, sl]
                        if B > 1:
                            si = idx3_v[b, 1, sl] * B + idx3_v[b, 2, sl]
                        else:
                            si = idx3_v[b, 1, sl]
                        if partition:
                            lo = c * half
                            owned = jnp.logical_and(si >= lo, si < lo + half)
                            trash = (1 - c) * (n_seg - 1)
                            si = jnp.where(owned, si, trash)
                        si_v[b, sl] = si
                return body
            guard(j, f)

        def issue_gather(j, b):
            def f(cid):
                def body():
                    pltpu.async_copy(table.at[gi_v.at[b]], rows_v.at[b],
                                     sem_g.at[b])
                return body
            guard(j, f)

        def wait_gather_scatter(j, b):
            def f(cid):
                def body():
                    pltpu.make_async_copy(table.at[gi_v.at[b]], rows_v.at[b],
                                          sem_g.at[b]).wait()
                    pltpu.sync_copy(rows_v.at[b], acc.at[si_v.at[b]], add=True)
                return body
            guard(j, f)

        issue_idx(0, 0)
        issue_idx(1, 1)

        @pl.loop(0, npair)
        def _pipe(jj):
            for b in (0, 1):
                j = jj * 2 + b
                wait_idx(j, b)
                compute_idx(j, b)
                issue_idx(j + 2, b)
                issue_gather(j, b)
                wait_gather_scatter(j - 1, 1 - b)

        plsc.subcore_barrier()

        if partition:
            @pl.loop(0, nzw)
            def _dump(j):
                z = s + j * 16

                @pl.when(z < nz)
                def _():
                    off = pl.multiple_of(c * half + z * _ZB, _ZB)
                    sl = pl.ds(off, _ZB)
                    pltpu.sync_copy(acc.at[sl], zero_v)

                    @pl.loop(0, _ZB)
                    def _relu(i):
                        zero_v[i] = jnp.maximum(zero_v[i], 0.0)

                    pltpu.sync_copy(zero_v, out.at[sl])
        else:
            @pl.loop(0, nzg)
            def _dump(jo):
                for g in range(8):
                    z = s + (jo * 8 + g) * 16

                    @pl.when(z < nz)
                    def _():
                        sl = pl.ds(pl.multiple_of(z * _ZB, _ZB), _ZB)
                        pltpu.async_copy(acc.at[sl], out.at[c].at[sl], sem_z)
                for g in range(8):
                    z = s + (jo * 8 + g) * 16

                    @pl.when(z < nz)
                    def _():
                        sl = pl.ds(pl.multiple_of(z * _ZB, _ZB), _ZB)
                        pltpu.make_async_copy(acc.at[sl], out.at[c].at[sl],
                                              sem_z).wait()

    return conv


def _make_sc_pool(n_child, n_par):
    assert n_child % _CH == 0 and n_par % _ZB == 0
    nch = n_child // _CH
    nchw = _cdiv(nch, 32)
    nr = n_par // _ZB
    nrw = _cdiv(nr, 16)

    @functools.partial(
        pl.kernel,
        out_type=jax.ShapeDtypeStruct((2, n_par, 16), _F32),
        mesh=_mesh,
        compiler_params=_sc_params,
        scratch_types=[
            pltpu.VMEM((_CH, 16), _F32),
            pltpu.VMEM((_CH, 16), _F32),
            pltpu.VMEM((_CH,), _I32),
            pltpu.VMEM((n_par, 16), _F32),
            pltpu.VMEM((_ZB, 16), _F32),
            pltpu.VMEM((_ZB, 16), _F32),
            pltpu.VMEM_SHARED((16, n_par, 16), _F32),
        ],
    )
    def pool(parts, parent, out, a_v, b_v, par_v, pacc, red_v, tmp_v, stage):
        c = lax.axis_index("c")
        s = lax.axis_index("s")
        w = s * 2 + c

        @pl.loop(0, n_par // 4)
        def _zero(i):
            for r in range(4):
                pacc[i * 4 + r] = jnp.zeros((16,), _F32)

        @pl.loop(0, nchw)
        def _chunks(it):
            cid = w + it * 32

            @pl.when(cid < nch)
            def _():
                base = pl.multiple_of(cid * _CH, _CH)
                pltpu.sync_copy(parts.at[0].at[pl.ds(base, _CH)], a_v)
                pltpu.sync_copy(parts.at[1].at[pl.ds(base, _CH)], b_v)
                pltpu.sync_copy(parent.at[pl.ds(base, _CH)], par_v)

                @pl.loop(0, _CH // 16)
                def _grp(g):
                    pvec = par_v[pl.ds(g * 16, 16)]
                    for j in range(16):
                        p = pvec[j]
                        i = g * 16 + j
                        v = a_v[i] + b_v[i]
                        pacc[p] = jnp.maximum(pacc[p], v)

        pltpu.sync_copy(pacc, stage.at[s])
        plsc.subcore_barrier()

        @pl.loop(0, nrw)
        def _red(j):
            z = s + j * 16

            @pl.when(z < nr)
            def _():
                sl = pl.ds(pl.multiple_of(z * _ZB, _ZB), _ZB)
                pltpu.sync_copy(stage.at[0].at[sl], red_v)
                for k in range(1, 16):
                    pltpu.sync_copy(stage.at[k].at[sl], tmp_v)

                    @pl.loop(0, _ZB // 4)
                    def _mx(i):
                        for r in range(4):
                            red_v[i * 4 + r] = jnp.maximum(
                                red_v[i * 4 + r], tmp_v[i * 4 + r])

                pltpu.sync_copy(red_v, out.at[c].at[sl])

    return pool


def _tc_mm(x, w, n_out):
    def body(x_ref, w_ref, o_ref):
        o_ref[...] = jnp.dot(x_ref[...], w_ref[...],
                             preferred_element_type=_F32)

    return pl.pallas_call(
        body,
        out_shape=jax.ShapeDtypeStruct((x.shape[0], n_out), _F32),
    )(x, w)


def _tc_max_mm(q, w, n_out):
    def body(q_ref, w_ref, o_ref):
        m = jnp.maximum(q_ref[0], q_ref[1])
        o_ref[...] = jnp.dot(m, w_ref[...], preferred_element_type=_F32)

    return pl.pallas_call(
        body,
        out_shape=jax.ShapeDtypeStruct((q.shape[1], n_out), _F32),
    )(q, w)


def _tc_final(p, w):
    def body(p_ref, w_ref, o_ref):
        z = jnp.dot(p_ref[0] + p_ref[1], w_ref[...],
                    preferred_element_type=_F32)
        o_ref[...] = jax.nn.sigmoid(z)

    return pl.pallas_call(
        body,
        out_shape=jax.ShapeDtypeStruct((p.shape[1], w.shape[1]), _F32),
    )(p, w)


_conv1 = _make_sc_conv(_E1, _N1P, 32, 1)
_conv2 = _make_sc_conv(_E2, _N2P, 32, 1)
_conv3 = _make_sc_conv(_EU1P, _N2P, _K2, 1)
_conv4 = _make_sc_conv(_EU2, _N1 * _K2, 1, _K2)
_pool1 = _make_sc_pool(_N1P, _N2P)
_pool2 = _make_sc_pool(_N2P, _N3P)


def _slot_pad(wr, k, cout):
    cin = wr.shape[0]
    w3 = wr.reshape(cin, k, cout)
    w3 = jnp.pad(w3, ((0, 0), (0, 32 - k), (0, 16 - cout)))
    return w3.reshape(cin, 512)


def kernel(x, edge_index1, kidx1, parent1, edge_index2, kidx2, parent2,
           src_u1, dst_u1, kidx_u1, src_u2, dst_u2, kidx_u2,
           W1, W2, Wt1, Wt2):
    W1r = _slot_pad(jnp.transpose(W1, (1, 0, 2)).reshape(_CIN, _K3 * 16),
                    _K3, 16)
    T1 = _tc_mm(x, W1r, 512).reshape(_N1 * 32, 16)
    e1 = jnp.concatenate([edge_index1[0], edge_index1[1], kidx1])
    P1 = _conv1(T1, e1)
    par1p = jnp.pad(parent1, (0, _N1P - _N1))
    Q1 = _pool1(P1, par1p)

    W2r = _slot_pad(jnp.transpose(W2, (1, 0, 2)).reshape(16, _K3 * 4), _K3, 4)
    T2 = _tc_max_mm(Q1, W2r, 512).reshape(_N2P * 32, 16)
    e2 = jnp.concatenate([edge_index2[0], edge_index2[1], kidx2])
    P2 = _conv2(T2, e2)
    par2p = jnp.pad(parent2, (0, _N2P - _N2))
    Q2 = _pool2(P2, par2p)

    Wt1p = jnp.pad(Wt1, ((0, 0), (0, 12), (0, 0)))
    Wt1r = jnp.transpose(Wt1p, (1, 0, 2)).reshape(16, _K2 * 16)
    T3 = _tc_max_mm(Q2, Wt1r, _K2 * 16).reshape(_N3P * _K2, 16)
    npad = _EU1P - _EU1
    eu1 = jnp.concatenate([
        jnp.pad(src_u1, (0, npad)),
        jnp.pad(dst_u1, (0, npad), constant_values=_N2),
        jnp.pad(kidx_u1, (0, npad)),
    ])
    P3 = _conv3(T3, eu1)

    def _relu_body(p_ref, o_ref):
        o_ref[...] = jnp.maximum(p_ref[0] + p_ref[1], 0.0)

    H3 = pl.pallas_call(
        _relu_body,
        out_shape=jax.ShapeDtypeStruct((_N2P, 16), _F32),
    )(P3)

    eu2 = jnp.concatenate([src_u2, dst_u2, kidx_u2])
    P4 = _conv4(H3, eu2)
    A4 = P4.reshape(2, _N1, _K2 * 16)
    Wt2r = Wt2.reshape(_K2 * 16, _COUT)
    return _tc_final(A4, Wt2r)


# speedup vs baseline: 17.8104x; 1.1427x over previous
import functools

import jax
import jax.numpy as jnp
from jax import lax
from jax.experimental import pallas as pl
from jax.experimental.pallas import tpu as pltpu
from jax.experimental.pallas import tpu_sc as plsc

_N1, _N2, _N3 = 10000, 2500, 625
_E1, _E2 = 320000, 80000
_EU1, _EU2 = 20000, 80000
_CIN, _COUT = 128, 128
_K3, _K2 = 27, 8

_F32 = jnp.float32
_I32 = jnp.int32

_CH = 128
_ZB = 128

_N1P = 10240
_N2P = 2560
_N3P = 640
_EU1P = 20096

_mesh = plsc.VectorSubcoreMesh(core_axis_name="c", subcore_axis_name="s")
_sc_params = pltpu.CompilerParams(use_tc_tiling_on_sc=False)


def _cdiv(a, b):
    return (a + b - 1) // b


def _make_sc_conv(E, n_seg, SLOT, B, partition=False):
    assert E % _CH == 0 and n_seg % _ZB == 0
    nch = E // _CH
    nw = 16 if partition else 32
    nchw = _cdiv(nch, nw)
    ngrp = _cdiv(nchw + 3, 4)
    half = n_seg // 2
    if partition:
        assert half % _ZB == 0
    nz = (half if partition else n_seg) // _ZB
    nzz = n_seg // _ZB
    nzw = _cdiv(nz, 16)
    nzzw = _cdiv(nzz, 16)
    nzg = _cdiv(nzzw, 8)
    out_shape = (n_seg, 16) if partition else (2, n_seg, 16)

    @functools.partial(
        pl.kernel,
        out_type=jax.ShapeDtypeStruct(out_shape, _F32),
        mesh=_mesh,
        compiler_params=_sc_params,
        scratch_types=[
            pltpu.VMEM((4, 3, _CH), _I32),
            pltpu.VMEM((4, _CH), _I32),
            pltpu.VMEM((4, _CH), _I32),
            pltpu.VMEM((4, _CH, 16), _F32),
            pltpu.VMEM((_ZB, 16), _F32),
            pltpu.VMEM_SHARED((n_seg, 16), _F32),
            pltpu.SemaphoreType.DMA((4,)),
            pltpu.SemaphoreType.DMA((4,)),
            pltpu.SemaphoreType.DMA,
        ],
    )
    def conv(table, e3, out, idx3_v, gi_v, si_v, rows_v, zero_v, acc,
             sem_i, sem_g, sem_z):
        c = lax.axis_index("c")
        s = lax.axis_index("s")
        w = (s * 2 + c) if not partition else s

        @pl.loop(0, _ZB)
        def _zb(i):
            zero_v[i] = jnp.zeros((16,), _F32)

        def guard(j, fn):
            cid = w + j * nw
            pl.when(jnp.logical_and(cid >= 0, cid < nch))(fn(cid))

        def issue_idx(j, b):
            def f(cid):
                def body():
                    base = pl.multiple_of(cid * _CH, _CH)
                    for r in range(3):
                        pltpu.async_copy(e3.at[pl.ds(r * E + base, _CH)],
                                         idx3_v.at[b, r], sem_i.at[b])
                return body
            guard(j, f)

        def wait_idx(j, b):
            def f(cid):
                def body():
                    for r in range(3):
                        pltpu.make_async_copy(e3.at[pl.ds(r * E, _CH)],
                                              idx3_v.at[b, r],
                                              sem_i.at[b]).wait()
                return body
            guard(j, f)

        def compute_idx(j, b):
            def f(cid):
                def body():
                    for i in range(_CH // 16):
                        sl = pl.ds(i * 16, 16)
                        if SLOT > 1:
                            gi_v[b, sl] = (idx3_v[b, 0, sl] * SLOT
                                           + idx3_v[b, 2, sl])
                        else:
                            gi_v[b, sl] = idx3_v[b, 0, sl]
                        if B > 1:
                            si = idx3_v[b, 1, sl] * B + idx3_v[b, 2, sl]
                        else:
                            si = idx3_v[b, 1, sl]
                        if partition:
                            lo = c * half
                            owned = jnp.logical_and(si >= lo, si < lo + half)
                            trash = (1 - c) * (n_seg - 1)
                            si = jnp.where(owned, si, trash)
                        si_v[b, sl] = si
                return body
            guard(j, f)

        def issue_gather(j, b):
            def f(cid):
                def body():
                    pltpu.async_copy(table.at[gi_v.at[b]], rows_v.at[b],
                                     sem_g.at[b])
                return body
            guard(j, f)

        def wait_gather_scatter(j, b):
            def f(cid):
                def body():
                    pltpu.make_async_copy(table.at[gi_v.at[b]], rows_v.at[b],
                                          sem_g.at[b]).wait()
                    pltpu.sync_copy(rows_v.at[b], acc.at[si_v.at[b]], add=True)
                return body
            guard(j, f)

        @pl.loop(0, nzg)
        def _zero1(jo):
            for g in range(8):
                z = s + (jo * 8 + g) * 16

                @pl.when(z < nzz)
                def _():
                    off = pl.multiple_of(z * _ZB, _ZB)
                    pltpu.async_copy(zero_v, acc.at[pl.ds(off, _ZB)], sem_z)

        for b in range(4):
            issue_idx(b, b)

        @pl.loop(0, nzg)
        def _zero2(jo):
            for g in range(8):
                z = s + (jo * 8 + g) * 16

                @pl.when(z < nzz)
                def _():
                    off = pl.multiple_of(z * _ZB, _ZB)
                    pltpu.make_async_copy(
                        zero_v, acc.at[pl.ds(off, _ZB)], sem_z).wait()

        plsc.subcore_barrier()

        @pl.loop(0, ngrp)
        def _pipe(jj):
            for b in range(4):
                j = jj * 4 + b
                wait_idx(j, b)
                compute_idx(j, b)
                issue_idx(j + 4, b)
                issue_gather(j, b)
                wait_gather_scatter(j - 3, (b + 1) % 4)

        plsc.subcore_barrier()

        if partition:
            @pl.loop(0, nzw)
            def _dump(j):
                z = s + j * 16

                @pl.when(z < nz)
                def _():
                    off = pl.multiple_of(c * half + z * _ZB, _ZB)
                    sl = pl.ds(off, _ZB)
                    pltpu.sync_copy(acc.at[sl], zero_v)

                    @pl.loop(0, _ZB)
                    def _relu(i):
                        zero_v[i] = jnp.maximum(zero_v[i], 0.0)

                    pltpu.sync_copy(zero_v, out.at[sl])
        else:
            @pl.loop(0, nzg)
            def _dump(jo):
                for g in range(8):
                    z = s + (jo * 8 + g) * 16

                    @pl.when(z < nz)
                    def _():
                        sl = pl.ds(pl.multiple_of(z * _ZB, _ZB), _ZB)
                        pltpu.async_copy(acc.at[sl], out.at[c].at[sl], sem_z)
                for g in range(8):
                    z = s + (jo * 8 + g) * 16

                    @pl.when(z < nz)
                    def _():
                        sl = pl.ds(pl.multiple_of(z * _ZB, _ZB), _ZB)
                        pltpu.make_async_copy(acc.at[sl], out.at[c].at[sl],
                                              sem_z).wait()

    return conv


def _make_sc_pool(n_child, n_par):
    assert n_child % _CH == 0 and n_par % _ZB == 0
    nch = n_child // _CH
    nchw = _cdiv(nch, 32)
    nr = n_par // _ZB
    nrw = _cdiv(nr, 16)

    @functools.partial(
        pl.kernel,
        out_type=jax.ShapeDtypeStruct((2, n_par, 16), _F32),
        mesh=_mesh,
        compiler_params=_sc_params,
        scratch_types=[
            pltpu.VMEM((_CH, 16), _F32),
            pltpu.VMEM((_CH, 16), _F32),
            pltpu.VMEM((_CH,), _I32),
            pltpu.VMEM((n_par, 16), _F32),
            pltpu.VMEM((_ZB, 16), _F32),
            pltpu.VMEM((_ZB, 16), _F32),
            pltpu.VMEM_SHARED((16, n_par, 16), _F32),
        ],
    )
    def pool(parts, parent, out, a_v, b_v, par_v, pacc, red_v, tmp_v, stage):
        c = lax.axis_index("c")
        s = lax.axis_index("s")
        w = s * 2 + c

        @pl.loop(0, n_par // 4)
        def _zero(i):
            for r in range(4):
                pacc[i * 4 + r] = jnp.zeros((16,), _F32)

        @pl.loop(0, nchw)
        def _chunks(it):
            cid = w + it * 32

            @pl.when(cid < nch)
            def _():
                base = pl.multiple_of(cid * _CH, _CH)
                pltpu.sync_copy(parts.at[0].at[pl.ds(base, _CH)], a_v)
                pltpu.sync_copy(parts.at[1].at[pl.ds(base, _CH)], b_v)
                pltpu.sync_copy(parent.at[pl.ds(base, _CH)], par_v)

                @pl.loop(0, _CH // 16)
                def _grp(g):
                    pvec = par_v[pl.ds(g * 16, 16)]
                    for j in range(16):
                        p = pvec[j]
                        i = g * 16 + j
                        v = a_v[i] + b_v[i]
                        pacc[p] = jnp.maximum(pacc[p], v)

        pltpu.sync_copy(pacc, stage.at[s])
        plsc.subcore_barrier()

        @pl.loop(0, nrw)
        def _red(j):
            z = s + j * 16

            @pl.when(z < nr)
            def _():
                sl = pl.ds(pl.multiple_of(z * _ZB, _ZB), _ZB)
                pltpu.sync_copy(stage.at[0].at[sl], red_v)
                for k in range(1, 16):
                    pltpu.sync_copy(stage.at[k].at[sl], tmp_v)

                    @pl.loop(0, _ZB // 4)
                    def _mx(i):
                        for r in range(4):
                            red_v[i * 4 + r] = jnp.maximum(
                                red_v[i * 4 + r], tmp_v[i * 4 + r])

                pltpu.sync_copy(red_v, out.at[c].at[sl])

    return pool


def _tc_mm(x, w, n_out):
    def body(x_ref, w_ref, o_ref):
        o_ref[...] = jnp.dot(x_ref[...], w_ref[...],
                             preferred_element_type=_F32)

    return pl.pallas_call(
        body,
        out_shape=jax.ShapeDtypeStruct((x.shape[0], n_out), _F32),
    )(x, w)


def _tc_max_mm(q, w, n_out):
    def body(q_ref, w_ref, o_ref):
        m = jnp.maximum(q_ref[0], q_ref[1])
        o_ref[...] = jnp.dot(m, w_ref[...], preferred_element_type=_F32)

    return pl.pallas_call(
        body,
        out_shape=jax.ShapeDtypeStruct((q.shape[1], n_out), _F32),
    )(q, w)


def _tc_final(p, w):
    def body(p_ref, w_ref, o_ref):
        z = jnp.dot(p_ref[0] + p_ref[1], w_ref[...],
                    preferred_element_type=_F32)
        o_ref[...] = jax.nn.sigmoid(z)

    return pl.pallas_call(
        body,
        out_shape=jax.ShapeDtypeStruct((p.shape[1], w.shape[1]), _F32),
    )(p, w)


_conv1 = _make_sc_conv(_E1, _N1P, 32, 1)
_conv2 = _make_sc_conv(_E2, _N2P, 32, 1)
_conv3 = _make_sc_conv(_EU1P, _N2P, _K2, 1)
_conv4 = _make_sc_conv(_EU2, _N1 * _K2, 1, _K2)
_pool1 = _make_sc_pool(_N1P, _N2P)
_pool2 = _make_sc_pool(_N2P, _N3P)


def _slot_pad(wr, k, cout):
    cin = wr.shape[0]
    w3 = wr.reshape(cin, k, cout)
    w3 = jnp.pad(w3, ((0, 0), (0, 32 - k), (0, 16 - cout)))
    return w3.reshape(cin, 512)


def kernel(x, edge_index1, kidx1, parent1, edge_index2, kidx2, parent2,
           src_u1, dst_u1, kidx_u1, src_u2, dst_u2, kidx_u2,
           W1, W2, Wt1, Wt2):
    W1r = _slot_pad(jnp.transpose(W1, (1, 0, 2)).reshape(_CIN, _K3 * 16),
                    _K3, 16)
    T1 = _tc_mm(x, W1r, 512).reshape(_N1 * 32, 16)
    e1 = jnp.concatenate([edge_index1[0], edge_index1[1], kidx1])
    P1 = _conv1(T1, e1)
    par1p = jnp.pad(parent1, (0, _N1P - _N1))
    Q1 = _pool1(P1, par1p)

    W2r = _slot_pad(jnp.transpose(W2, (1, 0, 2)).reshape(16, _K3 * 4), _K3, 4)
    T2 = _tc_max_mm(Q1, W2r, 512).reshape(_N2P * 32, 16)
    e2 = jnp.concatenate([edge_index2[0], edge_index2[1], kidx2])
    P2 = _conv2(T2, e2)
    par2p = jnp.pad(parent2, (0, _N2P - _N2))
    Q2 = _pool2(P2, par2p)

    Wt1p = jnp.pad(Wt1, ((0, 0), (0, 12), (0, 0)))
    Wt1r = jnp.transpose(Wt1p, (1, 0, 2)).reshape(16, _K2 * 16)
    T3 = _tc_max_mm(Q2, Wt1r, _K2 * 16).reshape(_N3P * _K2, 16)
    npad = _EU1P - _EU1
    eu1 = jnp.concatenate([
        jnp.pad(src_u1, (0, npad)),
        jnp.pad(dst_u1, (0, npad), constant_values=_N2),
        jnp.pad(kidx_u1, (0, npad)),
    ])
    P3 = _conv3(T3, eu1)

    def _relu_body(p_ref, o_ref):
        o_ref[...] = jnp.maximum(p_ref[0] + p_ref[1], 0.0)

    H3 = pl.pallas_call(
        _relu_body,
        out_shape=jax.ShapeDtypeStruct((_N2P, 16), _F32),
    )(P3)

    eu2 = jnp.concatenate([src_u2, dst_u2, kidx_u2])
    P4 = _conv4(H3, eu2)
    A4 = P4.reshape(2, _N1, _K2 * 16)
    Wt2r = Wt2.reshape(_K2 * 16, _COUT)
    return _tc_final(A4, Wt2r)


# speedup vs baseline: 17.8948x; 1.0047x over previous
import functools

import jax
import jax.numpy as jnp
from jax import lax
from jax.experimental import pallas as pl
from jax.experimental.pallas import tpu as pltpu
from jax.experimental.pallas import tpu_sc as plsc

_N1, _N2, _N3 = 10000, 2500, 625
_E1, _E2 = 320000, 80000
_EU1, _EU2 = 20000, 80000
_CIN, _COUT = 128, 128
_K3, _K2 = 27, 8

_F32 = jnp.float32
_I32 = jnp.int32

_CH = 128
_ZB = 128

_N1P = 10240
_N2P = 2560
_N3P = 640
_EU1P = 20096

_mesh = plsc.VectorSubcoreMesh(core_axis_name="c", subcore_axis_name="s")
_sc_params = pltpu.CompilerParams(use_tc_tiling_on_sc=False)


def _cdiv(a, b):
    return (a + b - 1) // b


def _make_sc_conv(E, n_seg, SLOT, B, partition=False):
    assert E % _CH == 0 and n_seg % _ZB == 0
    nch = E // _CH
    nw = 16 if partition else 32
    nchw = _cdiv(nch, nw)
    ngrp = _cdiv(nchw + 3, 4)
    half = n_seg // 2
    if partition:
        assert half % _ZB == 0
    nz = (half if partition else n_seg) // _ZB
    nzz = n_seg // _ZB
    nzw = _cdiv(nz, 16)
    nzzw = _cdiv(nzz, 16)
    nzg = _cdiv(nzzw, 8)
    out_shape = (n_seg, 16) if partition else (2, n_seg, 16)

    @functools.partial(
        pl.kernel,
        out_type=jax.ShapeDtypeStruct(out_shape, _F32),
        mesh=_mesh,
        compiler_params=_sc_params,
        scratch_types=[
            pltpu.VMEM((4, 3, _CH), _I32),
            pltpu.VMEM((4, _CH), _I32),
            pltpu.VMEM((4, _CH), _I32),
            pltpu.VMEM((4, _CH, 16), _F32),
            pltpu.VMEM((_ZB, 16), _F32),
            pltpu.VMEM_SHARED((n_seg, 16), _F32),
            pltpu.SemaphoreType.DMA((4,)),
            pltpu.SemaphoreType.DMA((4,)),
            pltpu.SemaphoreType.DMA,
        ],
    )
    def conv(table, e3, out, idx3_v, gi_v, si_v, rows_v, zero_v, acc,
             sem_i, sem_g, sem_z):
        c = lax.axis_index("c")
        s = lax.axis_index("s")
        w = (s * 2 + c) if not partition else s

        @pl.loop(0, _ZB)
        def _zb(i):
            zero_v[i] = jnp.zeros((16,), _F32)

        def guard(j, fn):
            cid = w + j * nw
            pl.when(jnp.logical_and(cid >= 0, cid < nch))(fn(cid))

        def issue_idx(j, b):
            def f(cid):
                def body():
                    base = pl.multiple_of(cid * _CH, _CH)
                    for r in range(3):
                        pltpu.async_copy(e3.at[pl.ds(r * E + base, _CH)],
                                         idx3_v.at[b, r], sem_i.at[b])
                return body
            guard(j, f)

        def wait_idx(j, b):
            def f(cid):
                def body():
                    for r in range(3):
                        pltpu.make_async_copy(e3.at[pl.ds(r * E, _CH)],
                                              idx3_v.at[b, r],
                                              sem_i.at[b]).wait()
                return body
            guard(j, f)

        def compute_idx(j, b):
            def f(cid):
                def body():
                    for i in range(_CH // 16):
                        sl = pl.ds(i * 16, 16)
                        if SLOT > 1:
                            gi_v[b, sl] = (idx3_v[b, 0, sl] * SLOT
                                           + idx3_v[b, 2, sl])
                        else:
                            gi_v[b, sl] = idx3_v[b, 0, sl]
                        if B > 1:
                            si = idx3_v[b, 1, sl] * B + idx3_v[b, 2, sl]
                        else:
                            si = idx3_v[b, 1, sl]
                        if partition:
                            lo = c * half
                            owned = jnp.logical_and(si >= lo, si < lo + half)
                            trash = (1 - c) * (n_seg - 1)
                            si = jnp.where(owned, si, trash)
                        si_v[b, sl] = si
                return body
            guard(j, f)

        def issue_gather(j, b):
            def f(cid):
                def body():
                    pltpu.async_copy(table.at[gi_v.at[b]], rows_v.at[b],
                                     sem_g.at[b])
                return body
            guard(j, f)

        def wait_gather_scatter(j, b):
            def f(cid):
                def body():
                    pltpu.make_async_copy(table.at[gi_v.at[b]], rows_v.at[b],
                                          sem_g.at[b]).wait()
                    pltpu.sync_copy(rows_v.at[b], acc.at[si_v.at[b]], add=True)
                return body
            guard(j, f)

        @pl.loop(0, nzg)
        def _zero1(jo):
            for g in range(8):
                z = s + (jo * 8 + g) * 16

                @pl.when(z < nzz)
                def _():
                    off = pl.multiple_of(z * _ZB, _ZB)
                    pltpu.async_copy(zero_v, acc.at[pl.ds(off, _ZB)], sem_z)

        for b in range(4):
            issue_idx(b, b)

        @pl.loop(0, nzg)
        def _zero2(jo):
            for g in range(8):
                z = s + (jo * 8 + g) * 16

                @pl.when(z < nzz)
                def _():
                    off = pl.multiple_of(z * _ZB, _ZB)
                    pltpu.make_async_copy(
                        zero_v, acc.at[pl.ds(off, _ZB)], sem_z).wait()

        plsc.subcore_barrier()

        @pl.loop(0, ngrp)
        def _pipe(jj):
            for b in range(4):
                j = jj * 4 + b
                wait_idx(j, b)
                compute_idx(j, b)
                issue_idx(j + 4, b)
                issue_gather(j, b)
                wait_gather_scatter(j - 3, (b + 1) % 4)

        plsc.subcore_barrier()

        if partition:
            @pl.loop(0, nzw)
            def _dump(j):
                z = s + j * 16

                @pl.when(z < nz)
                def _():
                    off = pl.multiple_of(c * half + z * _ZB, _ZB)
                    sl = pl.ds(off, _ZB)
                    pltpu.sync_copy(acc.at[sl], zero_v)

                    @pl.loop(0, _ZB)
                    def _relu(i):
                        zero_v[i] = jnp.maximum(zero_v[i], 0.0)

                    pltpu.sync_copy(zero_v, out.at[sl])
        else:
            @pl.loop(0, nzg)
            def _dump(jo):
                for g in range(8):
                    z = s + (jo * 8 + g) * 16

                    @pl.when(z < nz)
                    def _():
                        sl = pl.ds(pl.multiple_of(z * _ZB, _ZB), _ZB)
                        pltpu.async_copy(acc.at[sl], out.at[c].at[sl], sem_z)
                for g in range(8):
                    z = s + (jo * 8 + g) * 16

                    @pl.when(z < nz)
                    def _():
                        sl = pl.ds(pl.multiple_of(z * _ZB, _ZB), _ZB)
                        pltpu.make_async_copy(acc.at[sl], out.at[c].at[sl],
                                              sem_z).wait()

    return conv


def _make_sc_pool(n_child, n_par):
    assert n_child % _CH == 0 and n_par % _ZB == 0
    nch = n_child // _CH
    nchw = _cdiv(nch, 32)
    nr = n_par // _ZB
    nrw = _cdiv(nr, 16)

    @functools.partial(
        pl.kernel,
        out_type=jax.ShapeDtypeStruct((2, n_par, 16), _F32),
        mesh=_mesh,
        compiler_params=_sc_params,
        scratch_types=[
            pltpu.VMEM((2, 2, _CH, 16), _F32),
            pltpu.VMEM((2, _CH), _I32),
            pltpu.VMEM((n_par, 16), _F32),
            pltpu.VMEM((_ZB, 16), _F32),
            pltpu.VMEM((2, _ZB, 16), _F32),
            pltpu.VMEM_SHARED((16, n_par, 16), _F32),
            pltpu.SemaphoreType.DMA((2,)),
            pltpu.SemaphoreType.DMA((2,)),
            pltpu.SemaphoreType.DMA((2,)),
        ],
    )
    def pool(parts, parent, out, ab_v, par_v, pacc, red_v, tmp_v, stage,
             sem_a, sem_p, sem_r):
        c = lax.axis_index("c")
        s = lax.axis_index("s")
        w = s * 2 + c

        @pl.loop(0, n_par // 8)
        def _zero(i):
            for r in range(8):
                pacc[i * 8 + r] = jnp.zeros((16,), _F32)

        def issue_chunk(j, b):
            cid = w + j * 32

            @pl.when(jnp.logical_and(cid >= 0, cid < nch))
            def _():
                base = pl.multiple_of(cid * _CH, _CH)
                pltpu.async_copy(parts.at[:, pl.ds(base, _CH)], ab_v.at[b],
                                 sem_a.at[b])
                pltpu.async_copy(parent.at[pl.ds(base, _CH)], par_v.at[b],
                                 sem_p.at[b])

        def process_chunk(j, b):
            cid = w + j * 32

            @pl.when(jnp.logical_and(cid >= 0, cid < nch))
            def _():
                pltpu.make_async_copy(parts.at[:, pl.ds(0, _CH)], ab_v.at[b],
                                      sem_a.at[b]).wait()
                pltpu.make_async_copy(parent.at[pl.ds(0, _CH)], par_v.at[b],
                                      sem_p.at[b]).wait()

                for g in range(_CH // 16):
                    pvec = par_v[b, pl.ds(g * 16, 16)]
                    for j16 in range(16):
                        p = pvec[j16]
                        i = g * 16 + j16
                        v = ab_v[b, 0, i] + ab_v[b, 1, i]
                        pacc[p] = jnp.maximum(pacc[p], v)

        issue_chunk(0, 0)
        issue_chunk(1, 1)

        @pl.loop(0, _cdiv(nchw, 2))
        def _chunks(jj):
            for b in (0, 1):
                j = jj * 2 + b
                process_chunk(j, b)
                issue_chunk(j + 2, b)

        pltpu.sync_copy(pacc, stage.at[s])
        plsc.subcore_barrier()

        @pl.loop(0, nrw)
        def _red(j):
            z = s + j * 16

            @pl.when(z < nr)
            def _():
                sl = pl.ds(pl.multiple_of(z * _ZB, _ZB), _ZB)
                pltpu.sync_copy(stage.at[0].at[sl], red_v)
                pltpu.async_copy(stage.at[1].at[sl], tmp_v.at[1], sem_r.at[1])
                for k in range(1, 16):
                    b = k % 2
                    pltpu.make_async_copy(stage.at[k].at[sl], tmp_v.at[b],
                                          sem_r.at[b]).wait()
                    if k < 15:
                        nb = (k + 1) % 2
                        pltpu.async_copy(stage.at[k + 1].at[sl], tmp_v.at[nb],
                                         sem_r.at[nb])

                    @pl.loop(0, _ZB // 8)
                    def _mx(i):
                        for r in range(8):
                            red_v[i * 8 + r] = jnp.maximum(
                                red_v[i * 8 + r], tmp_v[b, i * 8 + r])

                pltpu.sync_copy(red_v, out.at[c].at[sl])

    return pool


def _tc_mm(x, w, n_out, blocks=10):
    m = x.shape[0]
    bm = m // blocks

    def body(x_ref, w_ref, o_ref):
        o_ref[...] = jnp.dot(x_ref[...], w_ref[...],
                             preferred_element_type=_F32)

    return pl.pallas_call(
        body,
        grid=(blocks,),
        in_specs=[
            pl.BlockSpec((bm, x.shape[1]), lambda i: (i, 0)),
            pl.BlockSpec((w.shape[0], n_out), lambda i: (0, 0)),
        ],
        out_specs=pl.BlockSpec((bm, n_out), lambda i: (i, 0)),
        out_shape=jax.ShapeDtypeStruct((m, n_out), _F32),
    )(x, w)


def _tc_max_mm(q, w, n_out):
    def body(q_ref, w_ref, o_ref):
        m = jnp.maximum(q_ref[0], q_ref[1])
        o_ref[...] = jnp.dot(m, w_ref[...], preferred_element_type=_F32)

    return pl.pallas_call(
        body,
        out_shape=jax.ShapeDtypeStruct((q.shape[1], n_out), _F32),
    )(q, w)


def _tc_final(p, w, blocks=10):
    m = p.shape[1]
    bm = m // blocks

    def body(p_ref, w_ref, o_ref):
        z = jnp.dot(p_ref[0] + p_ref[1], w_ref[...],
                    preferred_element_type=_F32)
        o_ref[...] = jax.nn.sigmoid(z)

    return pl.pallas_call(
        body,
        grid=(blocks,),
        in_specs=[
            pl.BlockSpec((2, bm, p.shape[2]), lambda i: (0, i, 0)),
            pl.BlockSpec(w.shape, lambda i: (0, 0)),
        ],
        out_specs=pl.BlockSpec((bm, w.shape[1]), lambda i: (i, 0)),
        out_shape=jax.ShapeDtypeStruct((m, w.shape[1]), _F32),
    )(p, w)


_conv1 = _make_sc_conv(_E1, _N1P, 32, 1)
_conv2 = _make_sc_conv(_E2, _N2P, 32, 1)
_conv3 = _make_sc_conv(_EU1P, _N2P, _K2, 1)
_conv4 = _make_sc_conv(_EU2, _N1 * _K2, 1, _K2)
_pool1 = _make_sc_pool(_N1P, _N2P)
_pool2 = _make_sc_pool(_N2P, _N3P)


def _slot_pad(wr, k, cout):
    cin = wr.shape[0]
    w3 = wr.reshape(cin, k, cout)
    w3 = jnp.pad(w3, ((0, 0), (0, 32 - k), (0, 16 - cout)))
    return w3.reshape(cin, 512)


def kernel(x, edge_index1, kidx1, parent1, edge_index2, kidx2, parent2,
           src_u1, dst_u1, kidx_u1, src_u2, dst_u2, kidx_u2,
           W1, W2, Wt1, Wt2):
    W1r = _slot_pad(jnp.transpose(W1, (1, 0, 2)).reshape(_CIN, _K3 * 16),
                    _K3, 16)
    T1 = _tc_mm(x, W1r, 512).reshape(_N1 * 32, 16)
    e1 = jnp.concatenate([edge_index1[0], edge_index1[1], kidx1])
    P1 = _conv1(T1, e1)
    par1p = jnp.pad(parent1, (0, _N1P - _N1))
    Q1 = _pool1(P1, par1p)

    W2r = _slot_pad(jnp.transpose(W2, (1, 0, 2)).reshape(16, _K3 * 4), _K3, 4)
    T2 = _tc_max_mm(Q1, W2r, 512).reshape(_N2P * 32, 16)
    e2 = jnp.concatenate([edge_index2[0], edge_index2[1], kidx2])
    P2 = _conv2(T2, e2)
    par2p = jnp.pad(parent2, (0, _N2P - _N2))
    Q2 = _pool2(P2, par2p)

    Wt1p = jnp.pad(Wt1, ((0, 0), (0, 12), (0, 0)))
    Wt1r = jnp.transpose(Wt1p, (1, 0, 2)).reshape(16, _K2 * 16)
    T3 = _tc_max_mm(Q2, Wt1r, _K2 * 16).reshape(_N3P * _K2, 16)
    npad = _EU1P - _EU1
    eu1 = jnp.concatenate([
        jnp.pad(src_u1, (0, npad)),
        jnp.pad(dst_u1, (0, npad), constant_values=_N2),
        jnp.pad(kidx_u1, (0, npad)),
    ])
    P3 = _conv3(T3, eu1)

    def _relu_body(p_ref, o_ref):
        o_ref[...] = jnp.maximum(p_ref[0] + p_ref[1], 0.0)

    H3 = pl.pallas_call(
        _relu_body,
        out_shape=jax.ShapeDtypeStruct((_N2P, 16), _F32),
    )(P3)

    eu2 = jnp.concatenate([src_u2, dst_u2, kidx_u2])
    P4 = _conv4(H3, eu2)
    A4 = P4.reshape(2, _N1, _K2 * 16)
    Wt2r = Wt2.reshape(_K2 * 16, _COUT)
    return _tc_final(A4, Wt2r)


# speedup vs baseline: 18.2423x; 1.0194x over previous
import functools

import jax
import jax.numpy as jnp
from jax import lax
from jax.experimental import pallas as pl
from jax.experimental.pallas import tpu as pltpu
from jax.experimental.pallas import tpu_sc as plsc

_N1, _N2, _N3 = 10000, 2500, 625
_E1, _E2 = 320000, 80000
_EU1, _EU2 = 20000, 80000
_CIN, _COUT = 128, 128
_K3, _K2 = 27, 8

_F32 = jnp.float32
_I32 = jnp.int32

_CH = 128
_ZB = 128

_N1P = 10240
_N2P = 2560
_N3P = 640
_EU1P = 20096

_mesh = plsc.VectorSubcoreMesh(core_axis_name="c", subcore_axis_name="s")
_sc_params = pltpu.CompilerParams(use_tc_tiling_on_sc=False)


def _cdiv(a, b):
    return (a + b - 1) // b


def _make_sc_conv(E, n_seg, SLOT, B, partition=False):
    assert E % _CH == 0 and n_seg % _ZB == 0
    nch = E // _CH
    nw = 16 if partition else 32
    nchw = _cdiv(nch, nw)
    ngrp = _cdiv(nchw + 3, 4)
    half = n_seg // 2
    if partition:
        assert half % _ZB == 0
    nz = (half if partition else n_seg) // _ZB
    nzz = n_seg // _ZB
    nzw = _cdiv(nz, 16)
    nzzw = _cdiv(nzz, 16)
    nzg = _cdiv(nzzw, 8)
    out_shape = (n_seg, 16) if partition else (2, n_seg, 16)

    @functools.partial(
        pl.kernel,
        out_type=jax.ShapeDtypeStruct(out_shape, _F32),
        mesh=_mesh,
        compiler_params=_sc_params,
        scratch_types=[
            pltpu.VMEM((4, 3, _CH), _I32),
            pltpu.VMEM((4, _CH), _I32),
            pltpu.VMEM((4, _CH), _I32),
            pltpu.VMEM((4, _CH, 16), _F32),
            pltpu.VMEM((_ZB, 16), _F32),
            pltpu.VMEM_SHARED((n_seg, 16), _F32),
            pltpu.SemaphoreType.DMA((4,)),
            pltpu.SemaphoreType.DMA((4,)),
            pltpu.SemaphoreType.DMA,
        ],
    )
    def conv(table, e3, out, idx3_v, gi_v, si_v, rows_v, zero_v, acc,
             sem_i, sem_g, sem_z):
        c = lax.axis_index("c")
        s = lax.axis_index("s")
        w = (s * 2 + c) if not partition else s

        @pl.loop(0, _ZB)
        def _zb(i):
            zero_v[i] = jnp.zeros((16,), _F32)

        def guard(j, fn):
            cid = w + j * nw
            pl.when(jnp.logical_and(cid >= 0, cid < nch))(fn(cid))

        def issue_idx(j, b):
            def f(cid):
                def body():
                    base = pl.multiple_of(cid * _CH, _CH)
                    for r in range(3):
                        pltpu.async_copy(e3.at[pl.ds(r * E + base, _CH)],
                                         idx3_v.at[b, r], sem_i.at[b])
                return body
            guard(j, f)

        def wait_idx(j, b):
            def f(cid):
                def body():
                    for r in range(3):
                        pltpu.make_async_copy(e3.at[pl.ds(r * E, _CH)],
                                              idx3_v.at[b, r],
                                              sem_i.at[b]).wait()
                return body
            guard(j, f)

        def compute_idx(j, b):
            def f(cid):
                def body():
                    for i in range(_CH // 16):
                        sl = pl.ds(i * 16, 16)
                        if SLOT > 1:
                            gi_v[b, sl] = (idx3_v[b, 0, sl] * SLOT
                                           + idx3_v[b, 2, sl])
                        else:
                            gi_v[b, sl] = idx3_v[b, 0, sl]
                        if B > 1:
                            si = idx3_v[b, 1, sl] * B + idx3_v[b, 2, sl]
                        else:
                            si = idx3_v[b, 1, sl]
                        if partition:
                            lo = c * half
                            oi = ((si >= lo) & (si < lo + half)).astype(_I32)
                            trash = (1 - c) * (n_seg - 1)
                            si = si * oi + trash * (1 - oi)
                        si_v[b, sl] = si
                return body
            guard(j, f)

        def issue_gather(j, b):
            def f(cid):
                def body():
                    pltpu.async_copy(table.at[gi_v.at[b]], rows_v.at[b],
                                     sem_g.at[b])
                return body
            guard(j, f)

        def wait_gather_scatter(j, b):
            def f(cid):
                def body():
                    pltpu.make_async_copy(table.at[gi_v.at[b]], rows_v.at[b],
                                          sem_g.at[b]).wait()
                    pltpu.sync_copy(rows_v.at[b], acc.at[si_v.at[b]], add=True)
                return body
            guard(j, f)

        @pl.loop(0, nzg)
        def _zero1(jo):
            for g in range(8):
                z = s + (jo * 8 + g) * 16

                @pl.when(z < nzz)
                def _():
                    off = pl.multiple_of(z * _ZB, _ZB)
                    pltpu.async_copy(zero_v, acc.at[pl.ds(off, _ZB)], sem_z)

        for b in range(4):
            issue_idx(b, b)

        @pl.loop(0, nzg)
        def _zero2(jo):
            for g in range(8):
                z = s + (jo * 8 + g) * 16

                @pl.when(z < nzz)
                def _():
                    off = pl.multiple_of(z * _ZB, _ZB)
                    pltpu.make_async_copy(
                        zero_v, acc.at[pl.ds(off, _ZB)], sem_z).wait()

        plsc.subcore_barrier()

        @pl.loop(0, ngrp)
        def _pipe(jj):
            for b in range(4):
                j = jj * 4 + b
                wait_idx(j, b)
                compute_idx(j, b)
                issue_idx(j + 4, b)
                issue_gather(j, b)
                wait_gather_scatter(j - 3, (b + 1) % 4)

        plsc.subcore_barrier()

        if partition:
            @pl.loop(0, nzw)
            def _dump(j):
                z = s + j * 16

                @pl.when(z < nz)
                def _():
                    off = pl.multiple_of(c * half + z * _ZB, _ZB)
                    sl = pl.ds(off, _ZB)
                    pltpu.sync_copy(acc.at[sl], zero_v)

                    @pl.loop(0, _ZB)
                    def _relu(i):
                        zero_v[i] = jnp.maximum(zero_v[i], 0.0)

                    pltpu.sync_copy(zero_v, out.at[sl])
        else:
            @pl.loop(0, nzg)
            def _dump(jo):
                for g in range(8):
                    z = s + (jo * 8 + g) * 16

                    @pl.when(z < nz)
                    def _():
                        sl = pl.ds(pl.multiple_of(z * _ZB, _ZB), _ZB)
                        pltpu.async_copy(acc.at[sl], out.at[c].at[sl], sem_z)
                for g in range(8):
                    z = s + (jo * 8 + g) * 16

                    @pl.when(z < nz)
                    def _():
                        sl = pl.ds(pl.multiple_of(z * _ZB, _ZB), _ZB)
                        pltpu.make_async_copy(acc.at[sl], out.at[c].at[sl],
                                              sem_z).wait()

    return conv


def _make_sc_pool(n_child, n_par):
    assert n_child % _CH == 0 and n_par % _ZB == 0
    nch = n_child // _CH
    nchw = _cdiv(nch, 32)
    nr = n_par // _ZB
    nrw = _cdiv(nr, 16)

    @functools.partial(
        pl.kernel,
        out_type=jax.ShapeDtypeStruct((2, n_par, 16), _F32),
        mesh=_mesh,
        compiler_params=_sc_params,
        scratch_types=[
            pltpu.VMEM((2, 2, _CH, 16), _F32),
            pltpu.VMEM((2, _CH), _I32),
            pltpu.VMEM((n_par, 16), _F32),
            pltpu.VMEM((_ZB, 16), _F32),
            pltpu.VMEM((2, _ZB, 16), _F32),
            pltpu.VMEM_SHARED((16, n_par, 16), _F32),
            pltpu.SemaphoreType.DMA((2,)),
            pltpu.SemaphoreType.DMA((2,)),
            pltpu.SemaphoreType.DMA((2,)),
        ],
    )
    def pool(parts, parent, out, ab_v, par_v, pacc, red_v, tmp_v, stage,
             sem_a, sem_p, sem_r):
        c = lax.axis_index("c")
        s = lax.axis_index("s")
        w = s * 2 + c

        @pl.loop(0, n_par // 8)
        def _zero(i):
            for r in range(8):
                pacc[i * 8 + r] = jnp.zeros((16,), _F32)

        def issue_chunk(j, b):
            cid = w + j * 32

            @pl.when(jnp.logical_and(cid >= 0, cid < nch))
            def _():
                base = pl.multiple_of(cid * _CH, _CH)
                pltpu.async_copy(parts.at[:, pl.ds(base, _CH)], ab_v.at[b],
                                 sem_a.at[b])
                pltpu.async_copy(parent.at[pl.ds(base, _CH)], par_v.at[b],
                                 sem_p.at[b])

        def process_chunk(j, b):
            cid = w + j * 32

            @pl.when(jnp.logical_and(cid >= 0, cid < nch))
            def _():
                pltpu.make_async_copy(parts.at[:, pl.ds(0, _CH)], ab_v.at[b],
                                      sem_a.at[b]).wait()
                pltpu.make_async_copy(parent.at[pl.ds(0, _CH)], par_v.at[b],
                                      sem_p.at[b]).wait()

                for g in range(_CH // 16):
                    pvec = par_v[b, pl.ds(g * 16, 16)]
                    for j16 in range(16):
                        p = pvec[j16]
                        i = g * 16 + j16
                        v = ab_v[b, 0, i] + ab_v[b, 1, i]
                        pacc[p] = jnp.maximum(pacc[p], v)

        issue_chunk(0, 0)
        issue_chunk(1, 1)

        @pl.loop(0, _cdiv(nchw, 2))
        def _chunks(jj):
            for b in (0, 1):
                j = jj * 2 + b
                process_chunk(j, b)
                issue_chunk(j + 2, b)

        pltpu.sync_copy(pacc, stage.at[s])
        plsc.subcore_barrier()

        @pl.loop(0, nrw)
        def _red(j):
            z = s + j * 16

            @pl.when(z < nr)
            def _():
                sl = pl.ds(pl.multiple_of(z * _ZB, _ZB), _ZB)
                pltpu.sync_copy(stage.at[0].at[sl], red_v)
                pltpu.async_copy(stage.at[1].at[sl], tmp_v.at[1], sem_r.at[1])
                for k in range(1, 16):
                    b = k % 2
                    pltpu.make_async_copy(stage.at[k].at[sl], tmp_v.at[b],
                                          sem_r.at[b]).wait()
                    if k < 15:
                        nb = (k + 1) % 2
                        pltpu.async_copy(stage.at[k + 1].at[sl], tmp_v.at[nb],
                                         sem_r.at[nb])

                    @pl.loop(0, _ZB // 8)
                    def _mx(i):
                        for r in range(8):
                            red_v[i * 8 + r] = jnp.maximum(
                                red_v[i * 8 + r], tmp_v[b, i * 8 + r])

                pltpu.sync_copy(red_v, out.at[c].at[sl])

    return pool


def _tc_mm(x, w, n_out):
    def body(x_ref, w_ref, o_ref):
        o_ref[...] = jnp.dot(x_ref[...], w_ref[...],
                             preferred_element_type=_F32)

    return pl.pallas_call(
        body,
        out_shape=jax.ShapeDtypeStruct((x.shape[0], n_out), _F32),
    )(x, w)


def _tc_max_mm(q, w, n_out):
    def body(q_ref, w_ref, o_ref):
        m = jnp.maximum(q_ref[0], q_ref[1])
        o_ref[...] = jnp.dot(m, w_ref[...], preferred_element_type=_F32)

    return pl.pallas_call(
        body,
        out_shape=jax.ShapeDtypeStruct((q.shape[1], n_out), _F32),
    )(q, w)


def _tc_final(p, w):
    def body(p_ref, w_ref, o_ref):
        z = jnp.dot(p_ref[0] + p_ref[1], w_ref[...],
                    preferred_element_type=_F32)
        o_ref[...] = jax.nn.sigmoid(z)

    return pl.pallas_call(
        body,
        out_shape=jax.ShapeDtypeStruct((p.shape[1], w.shape[1]), _F32),
    )(p, w)


_conv1 = _make_sc_conv(_E1, _N1P, 32, 1)
_conv2 = _make_sc_conv(_E2, _N2P, 32, 1)
_conv3 = _make_sc_conv(_EU1P, _N2P, _K2, 1)
_conv4 = _make_sc_conv(_EU2, _N1 * _K2, 1, _K2)
_pool1 = _make_sc_pool(_N1P, _N2P)
_pool2 = _make_sc_pool(_N2P, _N3P)


def _slot_pad(wr, k, cout):
    cin = wr.shape[0]
    w3 = wr.reshape(cin, k, cout)
    w3 = jnp.pad(w3, ((0, 0), (0, 32 - k), (0, 16 - cout)))
    return w3.reshape(cin, 512)


def kernel(x, edge_index1, kidx1, parent1, edge_index2, kidx2, parent2,
           src_u1, dst_u1, kidx_u1, src_u2, dst_u2, kidx_u2,
           W1, W2, Wt1, Wt2):
    W1r = _slot_pad(jnp.transpose(W1, (1, 0, 2)).reshape(_CIN, _K3 * 16),
                    _K3, 16)
    T1 = _tc_mm(x, W1r, 512).reshape(_N1 * 32, 16)
    e1 = jnp.concatenate([edge_index1[0], edge_index1[1], kidx1])
    P1 = _conv1(T1, e1)
    par1p = jnp.pad(parent1, (0, _N1P - _N1))
    Q1 = _pool1(P1, par1p)

    W2r = _slot_pad(jnp.transpose(W2, (1, 0, 2)).reshape(16, _K3 * 4), _K3, 4)
    T2 = _tc_max_mm(Q1, W2r, 512).reshape(_N2P * 32, 16)
    e2 = jnp.concatenate([edge_index2[0], edge_index2[1], kidx2])
    P2 = _conv2(T2, e2)
    par2p = jnp.pad(parent2, (0, _N2P - _N2))
    Q2 = _pool2(P2, par2p)

    Wt1p = jnp.pad(Wt1, ((0, 0), (0, 12), (0, 0)))
    Wt1r = jnp.transpose(Wt1p, (1, 0, 2)).reshape(16, _K2 * 16)
    T3 = _tc_max_mm(Q2, Wt1r, _K2 * 16).reshape(_N3P * _K2, 16)
    npad = _EU1P - _EU1
    eu1 = jnp.concatenate([
        jnp.pad(src_u1, (0, npad)),
        jnp.pad(dst_u1, (0, npad), constant_values=_N2),
        jnp.pad(kidx_u1, (0, npad)),
    ])
    P3 = _conv3(T3, eu1)

    def _relu_body(p_ref, o_ref):
        o_ref[...] = jnp.maximum(p_ref[0] + p_ref[1], 0.0)

    H3 = pl.pallas_call(
        _relu_body,
        out_shape=jax.ShapeDtypeStruct((_N2P, 16), _F32),
    )(P3)

    eu2 = jnp.concatenate([src_u2, dst_u2, kidx_u2])
    P4 = _conv4(H3, eu2)
    A4 = P4.reshape(2, _N1, _K2 * 16)
    Wt2r = Wt2.reshape(_K2 * 16, _COUT)
    return _tc_final(A4, Wt2r)


# speedup vs baseline: 18.3175x; 1.0041x over previous
import functools

import jax
import jax.numpy as jnp
from jax import lax
from jax.experimental import pallas as pl
from jax.experimental.pallas import tpu as pltpu
from jax.experimental.pallas import tpu_sc as plsc

_N1, _N2, _N3 = 10000, 2500, 625
_E1, _E2 = 320000, 80000
_EU1, _EU2 = 20000, 80000
_CIN, _COUT = 128, 128
_K3, _K2 = 27, 8

_F32 = jnp.float32
_I32 = jnp.int32

_CH = 128
_ZB = 128

_N1P = 10240
_N2P = 2560
_N3P = 640
_EU1P = 20096

_mesh = plsc.VectorSubcoreMesh(core_axis_name="c", subcore_axis_name="s")
_sc_params = pltpu.CompilerParams(use_tc_tiling_on_sc=False)


def _cdiv(a, b):
    return (a + b - 1) // b


def _make_sc_conv(E, n_seg, SLOT, B, partition=False):
    assert E % _CH == 0 and n_seg % _ZB == 0
    nch = E // _CH
    nw = 16 if partition else 32
    nchw = _cdiv(nch, nw)
    ngrp = _cdiv(nchw + 4, 4)
    half = n_seg // 2
    if partition:
        assert half % _ZB == 0
    nz = (half if partition else n_seg) // _ZB
    nzz = n_seg // _ZB
    nzw = _cdiv(nz, 16)
    nzzw = _cdiv(nzz, 16)
    nzg = _cdiv(nzzw, 8)
    out_shape = (n_seg, 16) if partition else (2, n_seg, 16)

    @functools.partial(
        pl.kernel,
        out_type=jax.ShapeDtypeStruct(out_shape, _F32),
        mesh=_mesh,
        compiler_params=_sc_params,
        scratch_types=[
            pltpu.VMEM((4, 3, _CH), _I32),
            pltpu.VMEM((4, _CH), _I32),
            pltpu.VMEM((4, _CH), _I32),
            pltpu.VMEM((4, _CH, 16), _F32),
            pltpu.VMEM((_ZB, 16), _F32),
            pltpu.VMEM_SHARED((n_seg, 16), _F32),
            pltpu.SemaphoreType.DMA((4,)),
            pltpu.SemaphoreType.DMA((4,)),
            pltpu.SemaphoreType.DMA((4,)),
            pltpu.SemaphoreType.DMA,
        ],
    )
    def conv(table, e3, out, idx3_v, gi_v, si_v, rows_v, zero_v, acc,
             sem_i, sem_g, sem_s, sem_z):
        c = lax.axis_index("c")
        s = lax.axis_index("s")
        w = (s * 2 + c) if not partition else s

        @pl.loop(0, _ZB)
        def _zb(i):
            zero_v[i] = jnp.zeros((16,), _F32)

        def guard(j, fn):
            cid = w + j * nw
            pl.when(jnp.logical_and(cid >= 0, cid < nch))(fn(cid))

        def issue_idx(j, b):
            def f(cid):
                def body():
                    base = pl.multiple_of(cid * _CH, _CH)
                    for r in range(3):
                        pltpu.async_copy(e3.at[pl.ds(r * E + base, _CH)],
                                         idx3_v.at[b, r], sem_i.at[b])
                return body
            guard(j, f)

        def wait_idx(j, b):
            def f(cid):
                def body():
                    for r in range(3):
                        pltpu.make_async_copy(e3.at[pl.ds(r * E, _CH)],
                                              idx3_v.at[b, r],
                                              sem_i.at[b]).wait()
                return body
            guard(j, f)

        def compute_idx(j, b):
            def f(cid):
                def body():
                    for i in range(_CH // 16):
                        sl = pl.ds(i * 16, 16)
                        if SLOT > 1:
                            gi_v[b, sl] = (idx3_v[b, 0, sl] * SLOT
                                           + idx3_v[b, 2, sl])
                        else:
                            gi_v[b, sl] = idx3_v[b, 0, sl]
                        if B > 1:
                            si = idx3_v[b, 1, sl] * B + idx3_v[b, 2, sl]
                        else:
                            si = idx3_v[b, 1, sl]
                        if partition:
                            lo = c * half
                            oi = ((si >= lo) & (si < lo + half)).astype(_I32)
                            trash = (1 - c) * (n_seg - 1)
                            si = si * oi + trash * (1 - oi)
                        si_v[b, sl] = si
                return body
            guard(j, f)

        def issue_gather(j, b):
            def f(cid):
                def body():
                    pltpu.async_copy(table.at[gi_v.at[b]], rows_v.at[b],
                                     sem_g.at[b])
                return body
            guard(j, f)

        def wait_gather_scatter(j, b):
            def f(cid):
                def body():
                    pltpu.make_async_copy(table.at[gi_v.at[b]], rows_v.at[b],
                                          sem_g.at[b]).wait()
                    pltpu.async_copy(rows_v.at[b], acc.at[si_v.at[b]],
                                     sem_s.at[b], add=True)
                return body
            guard(j, f)

        def wait_scatter(j, b):
            def f(cid):
                def body():
                    pltpu.make_async_copy(rows_v.at[b], acc.at[si_v.at[b]],
                                          sem_s.at[b]).wait()
                return body
            guard(j, f)

        @pl.loop(0, nzg)
        def _zero1(jo):
            for g in range(8):
                z = s + (jo * 8 + g) * 16

                @pl.when(z < nzz)
                def _():
                    off = pl.multiple_of(z * _ZB, _ZB)
                    pltpu.async_copy(zero_v, acc.at[pl.ds(off, _ZB)], sem_z)

        for b in range(4):
            issue_idx(b, b)

        @pl.loop(0, nzg)
        def _zero2(jo):
            for g in range(8):
                z = s + (jo * 8 + g) * 16

                @pl.when(z < nzz)
                def _():
                    off = pl.multiple_of(z * _ZB, _ZB)
                    pltpu.make_async_copy(
                        zero_v, acc.at[pl.ds(off, _ZB)], sem_z).wait()

        plsc.subcore_barrier()

        @pl.loop(0, ngrp)
        def _pipe(jj):
            for b in range(4):
                j = jj * 4 + b
                wait_idx(j, b)
                wait_scatter(j - 4, b)
                compute_idx(j, b)
                issue_idx(j + 4, b)
                issue_gather(j, b)
                wait_gather_scatter(j - 3, (b + 1) % 4)

        plsc.subcore_barrier()

        if partition:
            @pl.loop(0, nzw)
            def _dump(j):
                z = s + j * 16

                @pl.when(z < nz)
                def _():
                    off = pl.multiple_of(c * half + z * _ZB, _ZB)
                    sl = pl.ds(off, _ZB)
                    pltpu.sync_copy(acc.at[sl], zero_v)

                    @pl.loop(0, _ZB)
                    def _relu(i):
                        zero_v[i] = jnp.maximum(zero_v[i], 0.0)

                    pltpu.sync_copy(zero_v, out.at[sl])
        else:
            @pl.loop(0, nzg)
            def _dump(jo):
                for g in range(8):
                    z = s + (jo * 8 + g) * 16

                    @pl.when(z < nz)
                    def _():
                        sl = pl.ds(pl.multiple_of(z * _ZB, _ZB), _ZB)
                        pltpu.async_copy(acc.at[sl], out.at[c].at[sl], sem_z)
                for g in range(8):
                    z = s + (jo * 8 + g) * 16

                    @pl.when(z < nz)
                    def _():
                        sl = pl.ds(pl.multiple_of(z * _ZB, _ZB), _ZB)
                        pltpu.make_async_copy(acc.at[sl], out.at[c].at[sl],
                                              sem_z).wait()

    return conv


def _make_sc_pool(n_child, n_par):
    assert n_child % _CH == 0 and n_par % _ZB == 0
    nch = n_child // _CH
    nchw = _cdiv(nch, 32)
    nr = n_par // _ZB
    nrw = _cdiv(nr, 16)

    @functools.partial(
        pl.kernel,
        out_type=jax.ShapeDtypeStruct((2, n_par, 16), _F32),
        mesh=_mesh,
        compiler_params=_sc_params,
        scratch_types=[
            pltpu.VMEM((2, 2, _CH, 16), _F32),
            pltpu.VMEM((2, _CH), _I32),
            pltpu.VMEM((n_par, 16), _F32),
            pltpu.VMEM((_ZB, 16), _F32),
            pltpu.VMEM((2, _ZB, 16), _F32),
            pltpu.VMEM_SHARED((16, n_par, 16), _F32),
            pltpu.SemaphoreType.DMA((2,)),
            pltpu.SemaphoreType.DMA((2,)),
            pltpu.SemaphoreType.DMA((2,)),
        ],
    )
    def pool(parts, parent, out, ab_v, par_v, pacc, red_v, tmp_v, stage,
             sem_a, sem_p, sem_r):
        c = lax.axis_index("c")
        s = lax.axis_index("s")
        w = s * 2 + c

        @pl.loop(0, n_par // 8)
        def _zero(i):
            for r in range(8):
                pacc[i * 8 + r] = jnp.zeros((16,), _F32)

        def issue_chunk(j, b):
            cid = w + j * 32

            @pl.when(jnp.logical_and(cid >= 0, cid < nch))
            def _():
                base = pl.multiple_of(cid * _CH, _CH)
                pltpu.async_copy(parts.at[:, pl.ds(base, _CH)], ab_v.at[b],
                                 sem_a.at[b])
                pltpu.async_copy(parent.at[pl.ds(base, _CH)], par_v.at[b],
                                 sem_p.at[b])

        def process_chunk(j, b):
            cid = w + j * 32

            @pl.when(jnp.logical_and(cid >= 0, cid < nch))
            def _():
                pltpu.make_async_copy(parts.at[:, pl.ds(0, _CH)], ab_v.at[b],
                                      sem_a.at[b]).wait()
                pltpu.make_async_copy(parent.at[pl.ds(0, _CH)], par_v.at[b],
                                      sem_p.at[b]).wait()

                for g in range(_CH // 16):
                    pvec = par_v[b, pl.ds(g * 16, 16)]
                    for j16 in range(16):
                        p = pvec[j16]
                        i = g * 16 + j16
                        v = ab_v[b, 0, i] + ab_v[b, 1, i]
                        pacc[p] = jnp.maximum(pacc[p], v)

        issue_chunk(0, 0)
        issue_chunk(1, 1)

        @pl.loop(0, _cdiv(nchw, 2))
        def _chunks(jj):
            for b in (0, 1):
                j = jj * 2 + b
                process_chunk(j, b)
                issue_chunk(j + 2, b)

        pltpu.sync_copy(pacc, stage.at[s])
        plsc.subcore_barrier()

        @pl.loop(0, nrw)
        def _red(j):
            z = s + j * 16

            @pl.when(z < nr)
            def _():
                sl = pl.ds(pl.multiple_of(z * _ZB, _ZB), _ZB)
                pltpu.sync_copy(stage.at[0].at[sl], red_v)
                pltpu.async_copy(stage.at[1].at[sl], tmp_v.at[1], sem_r.at[1])
                for k in range(1, 16):
                    b = k % 2
                    pltpu.make_async_copy(stage.at[k].at[sl], tmp_v.at[b],
                                          sem_r.at[b]).wait()
                    if k < 15:
                        nb = (k + 1) % 2
                        pltpu.async_copy(stage.at[k + 1].at[sl], tmp_v.at[nb],
                                         sem_r.at[nb])

                    @pl.loop(0, _ZB // 8)
                    def _mx(i):
                        for r in range(8):
                            red_v[i * 8 + r] = jnp.maximum(
                                red_v[i * 8 + r], tmp_v[b, i * 8 + r])

                pltpu.sync_copy(red_v, out.at[c].at[sl])

    return pool


def _tc_mm(x, w, n_out):
    def body(x_ref, w_ref, o_ref):
        o_ref[...] = jnp.dot(x_ref[...], w_ref[...],
                             preferred_element_type=_F32)

    return pl.pallas_call(
        body,
        out_shape=jax.ShapeDtypeStruct((x.shape[0], n_out), _F32),
    )(x, w)


def _tc_max_mm(q, w, n_out):
    def body(q_ref, w_ref, o_ref):
        m = jnp.maximum(q_ref[0], q_ref[1])
        o_ref[...] = jnp.dot(m, w_ref[...], preferred_element_type=_F32)

    return pl.pallas_call(
        body,
        out_shape=jax.ShapeDtypeStruct((q.shape[1], n_out), _F32),
    )(q, w)


def _tc_final(p, w):
    def body(p_ref, w_ref, o_ref):
        z = jnp.dot(p_ref[0] + p_ref[1], w_ref[...],
                    preferred_element_type=_F32)
        o_ref[...] = jax.nn.sigmoid(z)

    return pl.pallas_call(
        body,
        out_shape=jax.ShapeDtypeStruct((p.shape[1], w.shape[1]), _F32),
    )(p, w)


_conv1 = _make_sc_conv(_E1, _N1P, 32, 1)
_conv2 = _make_sc_conv(_E2, _N2P, 32, 1)
_conv3 = _make_sc_conv(_EU1P, _N2P, _K2, 1)
_conv4 = _make_sc_conv(_EU2, _N1 * _K2, 1, _K2)
_pool1 = _make_sc_pool(_N1P, _N2P)
_pool2 = _make_sc_pool(_N2P, _N3P)


def _slot_pad(wr, k, cout):
    cin = wr.shape[0]
    w3 = wr.reshape(cin, k, cout)
    w3 = jnp.pad(w3, ((0, 0), (0, 32 - k), (0, 16 - cout)))
    return w3.reshape(cin, 512)


def kernel(x, edge_index1, kidx1, parent1, edge_index2, kidx2, parent2,
           src_u1, dst_u1, kidx_u1, src_u2, dst_u2, kidx_u2,
           W1, W2, Wt1, Wt2):
    W1r = _slot_pad(jnp.transpose(W1, (1, 0, 2)).reshape(_CIN, _K3 * 16),
                    _K3, 16)
    T1 = _tc_mm(x, W1r, 512).reshape(_N1 * 32, 16)
    e1 = jnp.concatenate([edge_index1[0], edge_index1[1], kidx1])
    P1 = _conv1(T1, e1)
    par1p = jnp.pad(parent1, (0, _N1P - _N1))
    Q1 = _pool1(P1, par1p)

    W2r = _slot_pad(jnp.transpose(W2, (1, 0, 2)).reshape(16, _K3 * 4), _K3, 4)
    T2 = _tc_max_mm(Q1, W2r, 512).reshape(_N2P * 32, 16)
    e2 = jnp.concatenate([edge_index2[0], edge_index2[1], kidx2])
    P2 = _conv2(T2, e2)
    par2p = jnp.pad(parent2, (0, _N2P - _N2))
    Q2 = _pool2(P2, par2p)

    Wt1p = jnp.pad(Wt1, ((0, 0), (0, 12), (0, 0)))
    Wt1r = jnp.transpose(Wt1p, (1, 0, 2)).reshape(16, _K2 * 16)
    T3 = _tc_max_mm(Q2, Wt1r, _K2 * 16).reshape(_N3P * _K2, 16)
    npad = _EU1P - _EU1
    eu1 = jnp.concatenate([
        jnp.pad(src_u1, (0, npad)),
        jnp.pad(dst_u1, (0, npad), constant_values=_N2),
        jnp.pad(kidx_u1, (0, npad)),
    ])
    P3 = _conv3(T3, eu1)

    def _relu_body(p_ref, o_ref):
        o_ref[...] = jnp.maximum(p_ref[0] + p_ref[1], 0.0)

    H3 = pl.pallas_call(
        _relu_body,
        out_shape=jax.ShapeDtypeStruct((_N2P, 16), _F32),
    )(P3)

    eu2 = jnp.concatenate([src_u2, dst_u2, kidx_u2])
    P4 = _conv4(H3, eu2)
    A4 = P4.reshape(2, _N1, _K2 * 16)
    Wt2r = Wt2.reshape(_K2 * 16, _COUT)
    return _tc_final(A4, Wt2r)


# speedup vs baseline: 18.7364x; 1.0229x over previous
import functools

import jax
import jax.numpy as jnp
from jax import lax
from jax.experimental import pallas as pl
from jax.experimental.pallas import tpu as pltpu
from jax.experimental.pallas import tpu_sc as plsc

_N1, _N2, _N3 = 10000, 2500, 625
_E1, _E2 = 320000, 80000
_EU1, _EU2 = 20000, 80000
_CIN, _COUT = 128, 128
_K3, _K2 = 27, 8

_F32 = jnp.float32
_I32 = jnp.int32

_CH = 128
_ZB = 128

_N1P = 10240
_N2P = 2560
_N3P = 640
_EU1P = 20096

_mesh = plsc.VectorSubcoreMesh(core_axis_name="c", subcore_axis_name="s")
_sc_params = pltpu.CompilerParams(use_tc_tiling_on_sc=False)


def _cdiv(a, b):
    return (a + b - 1) // b


def _make_sc_conv(E, n_seg, SLOT, B, partition=False):
    assert E % _CH == 0 and n_seg % _ZB == 0
    nch = E // _CH
    nw = 16 if partition else 32
    nchw = _cdiv(nch, nw)
    ngrp = _cdiv(nchw + 6, 6)
    half = n_seg // 2
    if partition:
        assert half % _ZB == 0
    nz = (half if partition else n_seg) // _ZB
    nzz = n_seg // _ZB
    nzw = _cdiv(nz, 16)
    nzzw = _cdiv(nzz, 16)
    nzg = _cdiv(nzzw, 8)
    out_shape = (n_seg, 16) if partition else (2, n_seg, 16)

    @functools.partial(
        pl.kernel,
        out_type=jax.ShapeDtypeStruct(out_shape, _F32),
        mesh=_mesh,
        compiler_params=_sc_params,
        scratch_types=[
            pltpu.VMEM((6, 3, _CH), _I32),
            pltpu.VMEM((6, _CH), _I32),
            pltpu.VMEM((6, _CH), _I32),
            pltpu.VMEM((6, _CH, 16), _F32),
            pltpu.VMEM((_ZB, 16), _F32),
            pltpu.VMEM_SHARED((n_seg, 16), _F32),
            pltpu.SemaphoreType.DMA((6,)),
            pltpu.SemaphoreType.DMA((6,)),
            pltpu.SemaphoreType.DMA((6,)),
            pltpu.SemaphoreType.DMA,
        ],
    )
    def conv(table, e3, out, idx3_v, gi_v, si_v, rows_v, zero_v, acc,
             sem_i, sem_g, sem_s, sem_z):
        c = lax.axis_index("c")
        s = lax.axis_index("s")
        w = (s * 2 + c) if not partition else s

        @pl.loop(0, _ZB)
        def _zb(i):
            zero_v[i] = jnp.zeros((16,), _F32)

        def guard(j, fn):
            cid = w + j * nw
            pl.when(jnp.logical_and(cid >= 0, cid < nch))(fn(cid))

        def issue_idx(j, b):
            def f(cid):
                def body():
                    base = pl.multiple_of(cid * _CH, _CH)
                    for r in range(3):
                        pltpu.async_copy(e3.at[pl.ds(r * E + base, _CH)],
                                         idx3_v.at[b, r], sem_i.at[b])
                return body
            guard(j, f)

        def wait_idx(j, b):
            def f(cid):
                def body():
                    for r in range(3):
                        pltpu.make_async_copy(e3.at[pl.ds(r * E, _CH)],
                                              idx3_v.at[b, r],
                                              sem_i.at[b]).wait()
                return body
            guard(j, f)

        def compute_idx(j, b):
            def f(cid):
                def body():
                    for i in range(_CH // 16):
                        sl = pl.ds(i * 16, 16)
                        if SLOT > 1:
                            gi_v[b, sl] = (idx3_v[b, 0, sl] * SLOT
                                           + idx3_v[b, 2, sl])
                        else:
                            gi_v[b, sl] = idx3_v[b, 0, sl]
                        if B > 1:
                            si = idx3_v[b, 1, sl] * B + idx3_v[b, 2, sl]
                        else:
                            si = idx3_v[b, 1, sl]
                        if partition:
                            lo = c * half
                            oi = ((si >= lo) & (si < lo + half)).astype(_I32)
                            trash = (1 - c) * (n_seg - 1)
                            si = si * oi + trash * (1 - oi)
                        si_v[b, sl] = si
                return body
            guard(j, f)

        def issue_gather(j, b):
            def f(cid):
                def body():
                    pltpu.async_copy(table.at[gi_v.at[b]], rows_v.at[b],
                                     sem_g.at[b])
                return body
            guard(j, f)

        def wait_gather_scatter(j, b):
            def f(cid):
                def body():
                    pltpu.make_async_copy(table.at[gi_v.at[b]], rows_v.at[b],
                                          sem_g.at[b]).wait()
                    pltpu.async_copy(rows_v.at[b], acc.at[si_v.at[b]],
                                     sem_s.at[b], add=True)
                return body
            guard(j, f)

        def wait_scatter(j, b):
            def f(cid):
                def body():
                    pltpu.make_async_copy(rows_v.at[b], acc.at[si_v.at[b]],
                                          sem_s.at[b]).wait()
                return body
            guard(j, f)

        @pl.loop(0, nzg)
        def _zero1(jo):
            for g in range(8):
                z = s + (jo * 8 + g) * 16

                @pl.when(z < nzz)
                def _():
                    off = pl.multiple_of(z * _ZB, _ZB)
                    pltpu.async_copy(zero_v, acc.at[pl.ds(off, _ZB)], sem_z)

        for b in range(6):
            issue_idx(b, b)

        @pl.loop(0, nzg)
        def _zero2(jo):
            for g in range(8):
                z = s + (jo * 8 + g) * 16

                @pl.when(z < nzz)
                def _():
                    off = pl.multiple_of(z * _ZB, _ZB)
                    pltpu.make_async_copy(
                        zero_v, acc.at[pl.ds(off, _ZB)], sem_z).wait()

        plsc.subcore_barrier()

        @pl.loop(0, ngrp)
        def _pipe(jj):
            for b in range(6):
                j = jj * 6 + b
                wait_idx(j, b)
                wait_scatter(j - 6, b)
                compute_idx(j, b)
                issue_idx(j + 6, b)
                issue_gather(j, b)
                wait_gather_scatter(j - 5, (b + 1) % 6)

        plsc.subcore_barrier()

        if partition:
            @pl.loop(0, nzw)
            def _dump(j):
                z = s + j * 16

                @pl.when(z < nz)
                def _():
                    off = pl.multiple_of(c * half + z * _ZB, _ZB)
                    sl = pl.ds(off, _ZB)
                    pltpu.sync_copy(acc.at[sl], zero_v)

                    @pl.loop(0, _ZB)
                    def _relu(i):
                        zero_v[i] = jnp.maximum(zero_v[i], 0.0)

                    pltpu.sync_copy(zero_v, out.at[sl])
        else:
            @pl.loop(0, nzg)
            def _dump(jo):
                for g in range(8):
                    z = s + (jo * 8 + g) * 16

                    @pl.when(z < nz)
                    def _():
                        sl = pl.ds(pl.multiple_of(z * _ZB, _ZB), _ZB)
                        pltpu.async_copy(acc.at[sl], out.at[c].at[sl], sem_z)
                for g in range(8):
                    z = s + (jo * 8 + g) * 16

                    @pl.when(z < nz)
                    def _():
                        sl = pl.ds(pl.multiple_of(z * _ZB, _ZB), _ZB)
                        pltpu.make_async_copy(acc.at[sl], out.at[c].at[sl],
                                              sem_z).wait()

    return conv


def _make_sc_pool(n_child, n_par):
    assert n_child % _CH == 0 and n_par % _ZB == 0
    nch = n_child // _CH
    nchw = _cdiv(nch, 32)
    nr = n_par // _ZB
    nrw = _cdiv(nr, 16)

    @functools.partial(
        pl.kernel,
        out_type=jax.ShapeDtypeStruct((2, n_par, 16), _F32),
        mesh=_mesh,
        compiler_params=_sc_params,
        scratch_types=[
            pltpu.VMEM((2, 2, _CH, 16), _F32),
            pltpu.VMEM((2, _CH), _I32),
            pltpu.VMEM((n_par, 16), _F32),
            pltpu.VMEM((_ZB, 16), _F32),
            pltpu.VMEM((2, _ZB, 16), _F32),
            pltpu.VMEM_SHARED((16, n_par, 16), _F32),
            pltpu.SemaphoreType.DMA((2,)),
            pltpu.SemaphoreType.DMA((2,)),
            pltpu.SemaphoreType.DMA((2,)),
        ],
    )
    def pool(parts, parent, out, ab_v, par_v, pacc, red_v, tmp_v, stage,
             sem_a, sem_p, sem_r):
        c = lax.axis_index("c")
        s = lax.axis_index("s")
        w = s * 2 + c

        @pl.loop(0, n_par // 8)
        def _zero(i):
            for r in range(8):
                pacc[i * 8 + r] = jnp.zeros((16,), _F32)

        def issue_chunk(j, b):
            cid = w + j * 32

            @pl.when(jnp.logical_and(cid >= 0, cid < nch))
            def _():
                base = pl.multiple_of(cid * _CH, _CH)
                pltpu.async_copy(parts.at[:, pl.ds(base, _CH)], ab_v.at[b],
                                 sem_a.at[b])
                pltpu.async_copy(parent.at[pl.ds(base, _CH)], par_v.at[b],
                                 sem_p.at[b])

        def process_chunk(j, b):
            cid = w + j * 32

            @pl.when(jnp.logical_and(cid >= 0, cid < nch))
            def _():
                pltpu.make_async_copy(parts.at[:, pl.ds(0, _CH)], ab_v.at[b],
                                      sem_a.at[b]).wait()
                pltpu.make_async_copy(parent.at[pl.ds(0, _CH)], par_v.at[b],
                                      sem_p.at[b]).wait()

                for g in range(_CH // 16):
                    pvec = par_v[b, pl.ds(g * 16, 16)]
                    for j16 in range(16):
                        p = pvec[j16]
                        i = g * 16 + j16
                        v = ab_v[b, 0, i] + ab_v[b, 1, i]
                        pacc[p] = jnp.maximum(pacc[p], v)

        issue_chunk(0, 0)
        issue_chunk(1, 1)

        @pl.loop(0, _cdiv(nchw, 2))
        def _chunks(jj):
            for b in (0, 1):
                j = jj * 2 + b
                process_chunk(j, b)
                issue_chunk(j + 2, b)

        pltpu.sync_copy(pacc, stage.at[s])
        plsc.subcore_barrier()

        @pl.loop(0, nrw)
        def _red(j):
            z = s + j * 16

            @pl.when(z < nr)
            def _():
                sl = pl.ds(pl.multiple_of(z * _ZB, _ZB), _ZB)
                pltpu.sync_copy(stage.at[0].at[sl], red_v)
                pltpu.async_copy(stage.at[1].at[sl], tmp_v.at[1], sem_r.at[1])
                for k in range(1, 16):
                    b = k % 2
                    pltpu.make_async_copy(stage.at[k].at[sl], tmp_v.at[b],
                                          sem_r.at[b]).wait()
                    if k < 15:
                        nb = (k + 1) % 2
                        pltpu.async_copy(stage.at[k + 1].at[sl], tmp_v.at[nb],
                                         sem_r.at[nb])

                    @pl.loop(0, _ZB // 8)
                    def _mx(i):
                        for r in range(8):
                            red_v[i * 8 + r] = jnp.maximum(
                                red_v[i * 8 + r], tmp_v[b, i * 8 + r])

                pltpu.sync_copy(red_v, out.at[c].at[sl])

    return pool


def _tc_mm(x, w, n_out):
    def body(x_ref, w_ref, o_ref):
        o_ref[...] = jnp.dot(x_ref[...], w_ref[...],
                             preferred_element_type=_F32)

    return pl.pallas_call(
        body,
        out_shape=jax.ShapeDtypeStruct((x.shape[0], n_out), _F32),
    )(x, w)


def _tc_max_mm(q, w, n_out):
    def body(q_ref, w_ref, o_ref):
        m = jnp.maximum(q_ref[0], q_ref[1])
        o_ref[...] = jnp.dot(m, w_ref[...], preferred_element_type=_F32)

    return pl.pallas_call(
        body,
        out_shape=jax.ShapeDtypeStruct((q.shape[1], n_out), _F32),
    )(q, w)


def _tc_final(p, w):
    def body(p_ref, w_ref, o_ref):
        z = jnp.dot(p_ref[0] + p_ref[1], w_ref[...],
                    preferred_element_type=_F32)
        o_ref[...] = jax.nn.sigmoid(z)

    return pl.pallas_call(
        body,
        out_shape=jax.ShapeDtypeStruct((p.shape[1], w.shape[1]), _F32),
    )(p, w)


_conv1 = _make_sc_conv(_E1, _N1P, 32, 1)
_conv2 = _make_sc_conv(_E2, _N2P, 32, 1)
_conv3 = _make_sc_conv(_EU1P, _N2P, _K2, 1)
_conv4 = _make_sc_conv(_EU2, _N1 * _K2, 1, _K2)
_pool1 = _make_sc_pool(_N1P, _N2P)
_pool2 = _make_sc_pool(_N2P, _N3P)


def _slot_pad(wr, k, cout):
    cin = wr.shape[0]
    w3 = wr.reshape(cin, k, cout)
    w3 = jnp.pad(w3, ((0, 0), (0, 32 - k), (0, 16 - cout)))
    return w3.reshape(cin, 512)


def kernel(x, edge_index1, kidx1, parent1, edge_index2, kidx2, parent2,
           src_u1, dst_u1, kidx_u1, src_u2, dst_u2, kidx_u2,
           W1, W2, Wt1, Wt2):
    W1r = _slot_pad(jnp.transpose(W1, (1, 0, 2)).reshape(_CIN, _K3 * 16),
                    _K3, 16)
    T1 = _tc_mm(x, W1r, 512).reshape(_N1 * 32, 16)
    e1 = jnp.concatenate([edge_index1[0], edge_index1[1], kidx1])
    P1 = _conv1(T1, e1)
    par1p = jnp.pad(parent1, (0, _N1P - _N1))
    Q1 = _pool1(P1, par1p)

    W2r = _slot_pad(jnp.transpose(W2, (1, 0, 2)).reshape(16, _K3 * 4), _K3, 4)
    T2 = _tc_max_mm(Q1, W2r, 512).reshape(_N2P * 32, 16)
    e2 = jnp.concatenate([edge_index2[0], edge_index2[1], kidx2])
    P2 = _conv2(T2, e2)
    par2p = jnp.pad(parent2, (0, _N2P - _N2))
    Q2 = _pool2(P2, par2p)

    Wt1p = jnp.pad(Wt1, ((0, 0), (0, 12), (0, 0)))
    Wt1r = jnp.transpose(Wt1p, (1, 0, 2)).reshape(16, _K2 * 16)
    T3 = _tc_max_mm(Q2, Wt1r, _K2 * 16).reshape(_N3P * _K2, 16)
    npad = _EU1P - _EU1
    eu1 = jnp.concatenate([
        jnp.pad(src_u1, (0, npad)),
        jnp.pad(dst_u1, (0, npad), constant_values=_N2),
        jnp.pad(kidx_u1, (0, npad)),
    ])
    P3 = _conv3(T3, eu1)

    def _relu_body(p_ref, o_ref):
        o_ref[...] = jnp.maximum(p_ref[0] + p_ref[1], 0.0)

    H3 = pl.pallas_call(
        _relu_body,
        out_shape=jax.ShapeDtypeStruct((_N2P, 16), _F32),
    )(P3)

    eu2 = jnp.concatenate([src_u2, dst_u2, kidx_u2])
    P4 = _conv4(H3, eu2)
    A4 = P4.reshape(2, _N1, _K2 * 16)
    Wt2r = Wt2.reshape(_K2 * 16, _COUT)
    return _tc_final(A4, Wt2r)
